# Initial kernel scaffold; baseline (speedup 1.0000x reference)
import jax
import jax.numpy as jnp
from jax.experimental import pallas as pl


def kernel(x, edge_index, edge_attr, batch, params, return_hidden):
    raise NotImplementedError("write your pallas kernel here")


# speedup vs baseline: 1.1035x; 1.1035x over previous
import jax
import jax.numpy as jnp
from jax.experimental import pallas as pl

N = 10000
E = 320000
B = 256
H_DIM = 96
HEADS = 8
HC = 12
NUM_LAYERS = 3
TASKS = ["normal", "mcc26", "mkl1"]


def _fold_att(lin, att):
    return jnp.einsum("dhc,hc->dh", lin.reshape(H_DIM, HEADS, HC), att)


def kernel(x, edge_index, edge_attr, batch, params, return_hidden):
    src, dst = edge_index[0], edge_index[1]

    A = [_fold_att(params["gat"][l]["lin_edge"], params["gat"][l]["att_edge"])
         for l in range(NUM_LAYERS)]
    S = [_fold_att(params["gat"][l]["lin"], params["gat"][l]["att_src"])
         for l in range(NUM_LAYERS)]
    D = [_fold_att(params["gat"][l]["lin"], params["gat"][l]["att_dst"])
         for l in range(NUM_LAYERS)]
    We, be = params["edge_emb"]["W"], params["edge_emb"]["b"]
    Ce = jnp.concatenate([We @ A[l] for l in range(NUM_LAYERS)], axis=1)
    cb = jnp.concatenate([be @ A[l] for l in range(NUM_LAYERS)])

    h = x @ params["node_emb"]["W"] + params["node_emb"]["b"]

    ae_all = edge_attr @ Ce + cb
    ones = jnp.ones((E,), jnp.float32)
    deg = jax.ops.segment_sum(ones, dst, num_segments=N)
    degc = jnp.maximum(deg, 1.0)
    ae_loop_all = jax.ops.segment_sum(ae_all, dst, num_segments=N) / degc[:, None]

    hidden = [h]
    alphas = []
    residual = h
    for i in range(NUM_LAYERS):
        g = params["gat"][i]
        ae = ae_all[:, i * HEADS:(i + 1) * HEADS]
        ae_loop = ae_loop_all[:, i * HEADS:(i + 1) * HEADS]
        zs = h @ jnp.concatenate([g["lin"], S[i], D[i]], axis=1)
        xs = zs[:, :H_DIM]
        a_src = zs[:, H_DIM:H_DIM + HEADS]
        a_dst = zs[:, H_DIM + HEADS:]
        M = jax.nn.leaky_relu(
            a_src.max(0) + a_dst.max(0)
            + jnp.maximum(ae.max(0), ae_loop.max(0)), 0.2)
        lg_e = jax.nn.leaky_relu(a_src[src] + a_dst[dst] + ae, 0.2)
        lg_n = jax.nn.leaky_relu(a_src + a_dst + ae_loop, 0.2)
        e_e = jnp.exp(lg_e - M)
        e_n = jnp.exp(lg_n - M)
        s = jax.ops.segment_sum(e_e, dst, num_segments=N) + e_n
        sc = s + 1e-16
        alpha_e = e_e / sc[dst]
        alpha_n = e_n / sc
        alphas.append(jnp.concatenate([alpha_e, alpha_n], axis=0))
        xs3 = xs.reshape(N, HEADS, HC)
        msg = jax.ops.segment_sum(xs3[src] * alpha_e[:, :, None], dst,
                                  num_segments=N)
        out = (msg + xs3 * alpha_n[:, :, None]).reshape(N, H_DIM) + g["bias"]
        out = (out - g["bn_rm"]) / jnp.sqrt(g["bn_rv"] + 1e-5) * g["bn_g"] + g["bn_b"]
        out = jax.nn.relu(out)
        if i > 0 and i % 2 == 0:
            out = out + residual
            residual = out
        h = out
        hidden.append(h)

    cnt = jax.ops.segment_sum(jnp.ones((N,), jnp.float32), batch, num_segments=B)
    cntc = jnp.maximum(cnt, 1.0)
    x_mean = jax.ops.segment_sum(h, batch, num_segments=B) / cntc[:, None]
    mx = jax.ops.segment_max(h, batch, num_segments=B)
    x_max = jnp.where(jnp.isfinite(mx), mx, 0.0)
    gate = jax.nn.relu(h @ params["gate1"]["W"] + params["gate1"]["b"])
    gate = (gate @ params["gate2"]["W"] + params["gate2"]["b"])[:, 0]
    Mg = gate.max()
    eg = jnp.exp(gate - Mg)
    sg = jax.ops.segment_sum(eg, batch, num_segments=B)
    x_att = jax.ops.segment_sum(eg[:, None] * h, batch, num_segments=B) \
        / (sg + 1e-16)[:, None]

    gr = jnp.concatenate([x_mean, x_max, x_att], axis=1)
    gr = jax.nn.relu(gr @ params["mlp1"]["W"] + params["mlp1"]["b"])
    gr = jax.nn.relu(gr @ params["mlp2"]["W"] + params["mlp2"]["b"])
    shared = jax.nn.relu(gr @ params["sf1"]["W"] + params["sf1"]["b"])
    shared = jax.nn.relu(shared @ params["sf2"]["W"] + params["sf2"]["b"])

    proj = []
    for i in range(NUM_LAYERS + 1):
        pool = jax.ops.segment_sum(hidden[i], batch, num_segments=B) / cntc[:, None]
        proj.append(jax.nn.relu(pool @ params["proj"][i]["W"]
                                + params["proj"][i]["b"]))

    m = params["mha"]
    f = (shared @ m["Wv"] + m["bv"]) @ m["Wo"] + m["bo"]

    preds, confs, uncs = [], [], []
    for task in TASKS:
        hp = params["head_" + task]
        hh = jax.nn.relu(f @ hp[0]["W"] + hp[0]["b"])
        hh = jax.nn.relu(hh @ hp[1]["W"] + hp[1]["b"])
        preds.append(jax.nn.sigmoid(hh @ hp[2]["W"] + hp[2]["b"]))
        cs = []
        for cp in params["conf_" + task]:
            hc = jax.nn.relu(f @ cp[0]["W"] + cp[0]["b"])
            hc = jax.nn.relu(hc @ cp[1]["W"] + cp[1]["b"])
            cs.append(jax.nn.sigmoid(hc @ cp[2]["W"] + cp[2]["b"]))
        cs = jnp.stack(cs)
        conf = cs.mean(axis=0)
        unc = jnp.std(cs, axis=0, ddof=1)
        confs.append(conf * (1.0 - unc * 0.5))
        uncs.append(unc)

    return (*preds, *proj, *confs, shared, *alphas, *uncs)


# speedup vs baseline: 25.4359x; 23.0496x over previous
import functools
import jax
import jax.numpy as jnp
from jax import lax
from jax.experimental import pallas as pl
from jax.experimental.pallas import tpu as pltpu, tpu_sc as plsc

N = 10000
E = 320000
B = 256
H_DIM = 96
HEADS = 8
HC = 12
NUM_LAYERS = 3
TASKS = ["normal", "mcc26", "mkl1"]

NC, NS, LN = 2, 16, 16
NW = NC * NS
NP = 10240
EP = 327680
EL = 344064
ET1 = EL // NW
GT1 = ET1 // 128
ETD = EP // NW
F32 = jnp.float32
I32 = jnp.int32

_MESH = plsc.VectorSubcoreMesh(core_axis_name="c", subcore_axis_name="s")
_SC_PARAMS = pltpu.CompilerParams(use_tc_tiling_on_sc=False)


_GDN = lax.GatherDimensionNumbers(offset_dims=(), collapsed_slice_dims=(0,),
                                  start_index_map=(0,))


def _vgather(vec, idx):
    return lax.gather(vec, idx[:, None], _GDN, (1,),
                      mode=lax.GatherScatterMode.PROMISE_IN_BOUNDS)


def _wid():
    return lax.axis_index("s") * NC + lax.axis_index("c")


def _sc_scatter_deg(dstf, ae64, zeros64):
    CW = 64
    C, KI = 1024, 8
    NCH = ETD // C
    RT = NP // NS

    @functools.partial(
        pl.kernel,
        out_type=jax.ShapeDtypeStruct((NC, NP, CW), F32),
        mesh=_MESH,
        compiler_params=_SC_PARAMS,
        scratch_types=[
            pltpu.VMEM((KI, 128), I32),
            pltpu.VMEM((C, CW), F32),
            pltpu.VMEM_SHARED((NP, CW), F32),
            pltpu.SemaphoreType.DMA,
        ],
    )
    def k(dst_hbm, ae_hbm, z_hbm, out_hbm, didx, vals, acc, sem):
        cid = lax.axis_index("c")
        sid = lax.axis_index("s")
        w = _wid()
        pltpu.sync_copy(z_hbm.at[pl.ds(sid * RT, RT)], acc.at[pl.ds(sid * RT, RT)])
        plsc.subcore_barrier()
        g0 = w * (ETD // 128)
        e0 = w * ETD
        for ch in range(NCH):
            pltpu.sync_copy(dst_hbm.at[pl.ds(g0 + ch * KI, KI)], didx)
            pltpu.sync_copy(ae_hbm.at[pl.ds(e0 + ch * C, C)], vals)
            ds_ = [pltpu.async_copy(vals.at[pl.ds(j * 128, 128)],
                                    acc.at[didx.at[j]], sem, add=True)
                   for j in range(KI)]
            for d in ds_:
                d.wait()
        plsc.subcore_barrier()
        pltpu.sync_copy(acc.at[pl.ds(sid * RT, RT)],
                        out_hbm.at[cid, pl.ds(sid * RT, RT)])

    return k(dstf, ae64, zeros64)


def _sc_pass1(srcf, dstf, asrc, adst, aef, m16, zeros16):
    C, KI = 896, 7
    NCH = ET1 // C
    RT = NP // NS

    @functools.partial(
        pl.kernel,
        out_type=(jax.ShapeDtypeStruct((EL, 16), F32),
                  jax.ShapeDtypeStruct((NC, NP, 16), F32)),
        mesh=_MESH,
        compiler_params=_SC_PARAMS,
        scratch_types=[
            pltpu.VMEM((KI, 128), I32),
            pltpu.VMEM((KI, 128), I32),
            pltpu.VMEM((C, 16), F32),
            pltpu.VMEM((C, 16), F32),
            pltpu.VMEM((C, 16), F32),
            pltpu.VMEM((C, 16), F32),
            pltpu.VMEM((16,), F32),
            pltpu.VMEM_SHARED((NP, 16), F32),
            pltpu.SemaphoreType.DMA,
            pltpu.SemaphoreType.DMA,
        ],
    )
    def k(src_hbm, dst_hbm, as_hbm, ad_hbm, ae_hbm, m_hbm, z_hbm,
          e_hbm, sout_hbm,
          sidx, didx, g1, g2, aev, ev, mv, acc, sem, sem2):
        cid = lax.axis_index("c")
        sid = lax.axis_index("s")
        w = _wid()
        pltpu.sync_copy(z_hbm.at[pl.ds(sid * RT, RT)], acc.at[pl.ds(sid * RT, RT)])
        pltpu.sync_copy(m_hbm, mv)
        plsc.subcore_barrier()
        lanes = lax.iota(I32, 16)
        mk = lanes < 8
        g0 = w * GT1
        e0 = w * ET1
        for ch in range(NCH):
            pltpu.sync_copy(src_hbm.at[pl.ds(g0 + ch * KI, KI)], sidx)
            pltpu.sync_copy(dst_hbm.at[pl.ds(g0 + ch * KI, KI)], didx)
            pltpu.sync_copy(ae_hbm.at[pl.ds(e0 + ch * C, C)], aev)
            ds_ = [pltpu.async_copy(as_hbm.at[sidx.at[j]],
                                    g1.at[pl.ds(j * 128, 128)], sem)
                   for j in range(KI)]
            ds_ += [pltpu.async_copy(ad_hbm.at[didx.at[j]],
                                     g2.at[pl.ds(j * 128, 128)], sem)
                    for j in range(KI)]
            for d in ds_:
                d.wait()
            mvv = mv[...]

            def row(r, _):
                z = g1[r, :] + g2[r, :] + aev[r, :]
                z = jnp.where(z >= 0.0, z, z * 0.2)
                e = jnp.exp(z - mvv)
                ev[r, :] = jnp.where(mk, e, 0.0)
                return 0

            lax.fori_loop(0, C, row, 0)
            pltpu.sync_copy(ev, e_hbm.at[pl.ds(e0 + ch * C, C)])
            ds_ = [pltpu.async_copy(ev.at[pl.ds(j * 128, 128)],
                                    acc.at[didx.at[j]], sem2, add=True)
                   for j in range(KI)]
            for d in ds_:
                d.wait()
        plsc.subcore_barrier()
        pltpu.sync_copy(acc.at[pl.ds(sid * RT, RT)],
                        sout_hbm.at[cid, pl.ds(sid * RT, RT)])

    return k(srcf, dstf, asrc, adst, aef, m16, zeros16)


def _sc_pass2(srcf, dstf, ehbm, schbm, xs, hmap, zeros96):
    C, KI = 384, 3
    NCH = ET1 // C
    RT = NP // NS

    @functools.partial(
        pl.kernel,
        out_type=(jax.ShapeDtypeStruct((EL * 16,), F32),
                  jax.ShapeDtypeStruct((NC, NP, H_DIM), F32)),
        mesh=_MESH,
        compiler_params=_SC_PARAMS,
        scratch_types=[
            pltpu.VMEM((KI, 128), I32),
            pltpu.VMEM((KI, 128), I32),
            pltpu.VMEM((C, 16), F32),
            pltpu.VMEM((C, 16), F32),
            pltpu.VMEM((C * 16,), F32),
            pltpu.VMEM((C, H_DIM), F32),
            pltpu.VMEM((6, 16), I32),
            pltpu.VMEM_SHARED((NP, H_DIM), F32),
            pltpu.SemaphoreType.DMA,
            pltpu.SemaphoreType.DMA,
        ],
    )
    def k(src_hbm, dst_hbm, e_hbm, s_hbm, xs_hbm, hm_hbm, z_hbm,
          a_hbm, mout_hbm,
          sidx, didx, sv, ev, av, xv, hmv, acc, sem, sem2):
        cid = lax.axis_index("c")
        sid = lax.axis_index("s")
        w = _wid()
        pltpu.sync_copy(z_hbm.at[pl.ds(sid * RT, RT)], acc.at[pl.ds(sid * RT, RT)])
        pltpu.sync_copy(hm_hbm, hmv)
        plsc.subcore_barrier()
        hms = [hmv[j, :] for j in range(6)]
        g0 = w * GT1
        e0 = w * ET1
        for ch in range(NCH):
            pltpu.sync_copy(src_hbm.at[pl.ds(g0 + ch * KI, KI)], sidx)
            pltpu.sync_copy(dst_hbm.at[pl.ds(g0 + ch * KI, KI)], didx)
            pltpu.sync_copy(e_hbm.at[pl.ds(e0 + ch * C, C)], ev)
            ds_ = [pltpu.async_copy(s_hbm.at[didx.at[j]],
                                    sv.at[pl.ds(j * 128, 128)], sem)
                   for j in range(KI)]
            ds_ += [pltpu.async_copy(xs_hbm.at[sidx.at[j]],
                                     xv.at[pl.ds(j * 128, 128)], sem)
                    for j in range(KI)]
            for d in ds_:
                d.wait()

            def mrow(r, _):
                a_r = ev[r, :] / sv[r, :]
                av[pl.ds(r * 16, 16)] = a_r
                for j in range(6):
                    g = _vgather(a_r, hms[j])
                    xv[r, pl.ds(j * 16, 16)] = xv[r, pl.ds(j * 16, 16)] * g
                return 0

            lax.fori_loop(0, C, mrow, 0)
            pltpu.sync_copy(av, a_hbm.at[pl.ds((e0 + ch * C) * 16, C * 16)])
            ds_ = [pltpu.async_copy(xv.at[pl.ds(j * 128, 128)],
                                    acc.at[didx.at[j]], sem2, add=True)
                   for j in range(KI)]
            for d in ds_:
                d.wait()
        plsc.subcore_barrier()
        pltpu.sync_copy(acc.at[pl.ds(sid * RT, RT)],
                        mout_hbm.at[cid, pl.ds(sid * RT, RT)])

    return k(srcf, dstf, ehbm, schbm, xs, hmap, zeros96)


def _sc_maxpool(h, batch_np, zeros96):
    RT = 400

    @functools.partial(
        pl.kernel,
        out_type=jax.ShapeDtypeStruct((NW, B, H_DIM), F32),
        mesh=_MESH,
        compiler_params=_SC_PARAMS,
        scratch_types=[
            pltpu.VMEM((RT, H_DIM), F32),
            pltpu.VMEM((RT,), I32),
            pltpu.VMEM((B, H_DIM), F32),
        ],
    )
    def k(h_hbm, b_hbm, z_hbm, out_hbm, hv, bv, acc):
        w = _wid()
        pltpu.sync_copy(z_hbm.at[pl.ds(0, B)], acc)

        @pl.when(w < 25)
        def _():
            pltpu.sync_copy(h_hbm.at[pl.ds(w * RT, RT)], hv)
            pltpu.sync_copy(b_hbm.at[pl.ds(w * RT, RT)], bv)

            def grp(g, _):
                bjv = bv[pl.ds(g * 16, 16)]
                for t in range(16):
                    b = bjv[t]
                    r = g * 16 + t
                    for j in range(6):
                        cur = acc[b, pl.ds(j * 16, 16)]
                        acc[b, pl.ds(j * 16, 16)] = jnp.maximum(
                            cur, hv[r, pl.ds(j * 16, 16)])
                return 0

            lax.fori_loop(0, RT // 16, grp, 0)

        pltpu.sync_copy(acc, out_hbm.at[w])

    return k(h, batch_np, zeros96)


def _mm(x, w, b, act=False, colmax=False, br=512):
    R, K = x.shape
    W = w.shape[1]
    nb = R // br

    def body(x_ref, w_ref, b_ref, o_ref, *mx):
        acc = jnp.dot(x_ref[...], w_ref[...], preferred_element_type=F32)
        acc = acc + b_ref[...]
        if act:
            acc = jnp.maximum(acc, 0.0)
        o_ref[...] = acc
        if colmax:
            mx[0][...] = jnp.max(acc, axis=0, keepdims=True)[None]

    outs = [jax.ShapeDtypeStruct((R, W), F32)]
    ospecs = [pl.BlockSpec((br, W), lambda i: (i, 0))]
    if colmax:
        outs.append(jax.ShapeDtypeStruct((nb, 1, W), F32))
        ospecs.append(pl.BlockSpec((1, 1, W), lambda i: (i, 0, 0)))
    res = pl.pallas_call(
        body,
        grid=(nb,),
        in_specs=[pl.BlockSpec((br, K), lambda i: (i, 0)),
                  pl.BlockSpec((K, W), lambda i: (0, 0)),
                  pl.BlockSpec((1, W), lambda i: (0, 0))],
        out_specs=ospecs if colmax else ospecs[0],
        out_shape=outs if colmax else outs[0],
    )(x, w, b.reshape(1, W))
    return res if colmax else (res,)


def _tc_degcomb(parts):
    br = 512
    nb = NP // br

    def body(p_ref, o_ref, mx_ref):
        t = p_ref[0] + p_ref[1]
        degc = jnp.maximum(t[:, 48:49], 1.0)
        o = t / degc
        o_ref[...] = o
        mx_ref[...] = jnp.max(o, axis=0, keepdims=True)[None]

    return pl.pallas_call(
        body,
        grid=(nb,),
        in_specs=[pl.BlockSpec((2, br, 64), lambda i: (0, i, 0))],
        out_specs=[pl.BlockSpec((br, 64), lambda i: (i, 0)),
                   pl.BlockSpec((1, 1, 64), lambda i: (i, 0, 0))],
        out_shape=[jax.ShapeDtypeStruct((NP, 64), F32),
                   jax.ShapeDtypeStruct((nb, 1, 64), F32)],
    )(parts)


def _tc_scomb(parts):
    br = 512
    nb = NP // br

    def body(p_ref, o_ref):
        o_ref[...] = p_ref[0] + p_ref[1] + 1e-16

    return pl.pallas_call(
        body,
        grid=(nb,),
        in_specs=[pl.BlockSpec((2, br, 16), lambda i: (0, i, 0))],
        out_specs=pl.BlockSpec((br, 16), lambda i: (i, 0)),
        out_shape=jax.ShapeDtypeStruct((NP, 16), F32),
    )(parts)


def _tc_post(parts, bias, bn_scale, bn_shift, res):
    br = 512
    nb = NP // br
    with_res = res is not None

    def body(p_ref, b_ref, s_ref, t_ref, *rest):
        if with_res:
            r_ref, o_ref = rest
        else:
            (o_ref,) = rest
        v = (p_ref[0] + p_ref[1] + b_ref[...]) * s_ref[...] + t_ref[...]
        v = jnp.maximum(v, 0.0)
        if with_res:
            v = v + r_ref[...]
        o_ref[...] = v

    in_specs = [pl.BlockSpec((2, br, H_DIM), lambda i: (0, i, 0)),
                pl.BlockSpec((1, H_DIM), lambda i: (0, 0)),
                pl.BlockSpec((1, H_DIM), lambda i: (0, 0)),
                pl.BlockSpec((1, H_DIM), lambda i: (0, 0))]
    args = [parts, bias.reshape(1, H_DIM), bn_scale.reshape(1, H_DIM),
            bn_shift.reshape(1, H_DIM)]
    if with_res:
        in_specs.append(pl.BlockSpec((br, H_DIM), lambda i: (i, 0)))
        args.append(res)
    return pl.pallas_call(
        body,
        grid=(nb,),
        in_specs=in_specs,
        out_specs=pl.BlockSpec((br, H_DIM), lambda i: (i, 0)),
        out_shape=jax.ShapeDtypeStruct((NP, H_DIM), F32),
    )(*args)


def _tc_pool(h0, h1, h2, h3, gv, mg, bidx):
    br = 512
    nb = NP // br

    def body(h0r, h1r, h2r, h3r, gr, mgr, br_, oa, ob):
        i = pl.program_id(0)
        oh = (br_[...] == lax.broadcasted_iota(I32, (br, B), 1)).astype(F32)
        e = jnp.exp(gr[...] - mgr[0, 0])
        ec = e[:, 0:1]
        vals = jnp.concatenate([h0r[...], h1r[...], h2r[...], h3r[...],
                                ec * h3r[...]], axis=1)
        pa = lax.dot_general(oh, vals, (((0,), (0,)), ((), ())),
                             preferred_element_type=F32)
        vals2 = jnp.concatenate([e, jnp.ones((br, 16), F32)], axis=1)
        pb = lax.dot_general(oh, vals2, (((0,), (0,)), ((), ())),
                             preferred_element_type=F32)

        @pl.when(i == 0)
        def _():
            oa[...] = pa
            ob[...] = pb

        @pl.when(i > 0)
        def _():
            oa[...] += pa
            ob[...] += pb

    return pl.pallas_call(
        body,
        grid=(nb,),
        in_specs=[pl.BlockSpec((br, H_DIM), lambda i: (i, 0)),
                  pl.BlockSpec((br, H_DIM), lambda i: (i, 0)),
                  pl.BlockSpec((br, H_DIM), lambda i: (i, 0)),
                  pl.BlockSpec((br, H_DIM), lambda i: (i, 0)),
                  pl.BlockSpec((br, 16), lambda i: (i, 0)),
                  pl.BlockSpec((1, 1), lambda i: (0, 0)),
                  pl.BlockSpec((br, 1), lambda i: (i, 0))],
        out_specs=[pl.BlockSpec((B, 480), lambda i: (0, 0)),
                   pl.BlockSpec((B, 32), lambda i: (0, 0))],
        out_shape=[jax.ShapeDtypeStruct((B, 480), F32),
                   jax.ShapeDtypeStruct((B, 32), F32)],
    )(h0, h1, h2, h3, gv, mg, bidx)


def _tc_readout(poolA, poolB, maxparts, Wm1, bm1, Wm2, bm2, Ws1, bs1, Ws2, bs2,
                Wpj, bpj, Wv, bv, Wo, bo, W1, b1, W2, b2, W3, b3):
    def body(pa, pb, mp, wm1, cm1, wm2, cm2, ws1, cs1, ws2, cs2,
             wpj, cpj, wv, cv, wo, co, w1, c1, w2, c2, w3, c3,
             outa, outs, outp):
        cnt = jnp.maximum(pb[:, 16:17], 1.0)
        sg = pb[:, 0:1] + 1e-16
        xm = mp[pl.ds(0, B), :]
        for kk in range(1, NW):
            xm = jnp.maximum(xm, mp[pl.ds(kk * B, B), :])
        x_mean = pa[:, 288:384] / cnt
        x_att = pa[:, 384:480] / sg
        gr = jnp.concatenate([x_mean, xm, x_att], axis=1)
        gr = jnp.maximum(jnp.dot(gr, wm1[...], preferred_element_type=F32) + cm1[...], 0.0)
        gr = jnp.maximum(jnp.dot(gr, wm2[...], preferred_element_type=F32) + cm2[...], 0.0)
        sh = jnp.maximum(jnp.dot(gr, ws1[...], preferred_element_type=F32) + cs1[...], 0.0)
        sh = jnp.maximum(jnp.dot(sh, ws2[...], preferred_element_type=F32) + cs2[...], 0.0)
        pools = pa[:, 0:384] / cnt
        pj = jnp.maximum(jnp.dot(pools, wpj[...], preferred_element_type=F32) + cpj[...], 0.0)
        f = jnp.dot(sh, wv[...], preferred_element_type=F32) + cv[...]
        f = jnp.dot(f, wo[...], preferred_element_type=F32) + co[...]
        h1 = jnp.maximum(jnp.dot(f, w1[...], preferred_element_type=F32) + c1[...], 0.0)
        h2 = jnp.maximum(jnp.dot(h1, w2[...], preferred_element_type=F32) + c2[...], 0.0)
        zl = jnp.dot(h2, w3[...], preferred_element_type=F32) + c3[...]
        z = 1.0 / (1.0 + jnp.exp(-zl))
        cols = []
        for t in range(3):
            cols.append(z[:, 4 * t:4 * t + 1])
        for t in range(3):
            a = z[:, 4 * t + 1:4 * t + 2]
            bb = z[:, 4 * t + 2:4 * t + 3]
            c = z[:, 4 * t + 3:4 * t + 4]
            m = (a + bb + c) / 3.0
            var = ((a - m) ** 2 + (bb - m) ** 2 + (c - m) ** 2) / 2.0
            unc = jnp.sqrt(var)
            cols.append(m * (1.0 - unc * 0.5))
            cols.append(unc)
        outa[...] = jnp.concatenate(cols + [jnp.zeros((B, 7), F32)], axis=1)
        outs[...] = sh
        outp[...] = pj

    full = lambda shp: pl.BlockSpec(shp, lambda: tuple(0 for _ in shp))
    args = [poolA, poolB, maxparts,
            Wm1, bm1.reshape(1, -1), Wm2, bm2.reshape(1, -1),
            Ws1, bs1.reshape(1, -1), Ws2, bs2.reshape(1, -1),
            Wpj, bpj.reshape(1, -1), Wv, bv.reshape(1, -1),
            Wo, bo.reshape(1, -1), W1, b1.reshape(1, -1),
            W2, b2.reshape(1, -1), W3, b3.reshape(1, -1)]
    return pl.pallas_call(
        body,
        in_specs=[full(a.shape) for a in args],
        out_specs=[full((B, 16)), full((B, H_DIM)), full((B, 512))],
        out_shape=[jax.ShapeDtypeStruct((B, 16), F32),
                   jax.ShapeDtypeStruct((B, H_DIM), F32),
                   jax.ShapeDtypeStruct((B, 512), F32)],
    )(*args)


def _fold_att(lin, att):
    return jnp.einsum("dhc,hc->dh", lin.reshape(H_DIM, HEADS, HC), att)


def kernel(x, edge_index, edge_attr, batch, params, return_hidden):
    src = edge_index[0].astype(I32)
    dst = edge_index[1].astype(I32)

    gats = params["gat"]
    A = [_fold_att(g["lin_edge"], g["att_edge"]) for g in gats]
    S = [_fold_att(g["lin"], g["att_src"]) for g in gats]
    D = [_fold_att(g["lin"], g["att_dst"]) for g in gats]
    We, be = params["edge_emb"]["W"], params["edge_emb"]["b"]
    Wp64 = jnp.zeros((3, 64), F32)
    bp64 = jnp.zeros((64,), F32)
    for l in range(3):
        Wp64 = Wp64.at[:, l * 16:l * 16 + 8].set(We @ A[l])
        bp64 = bp64.at[l * 16:l * 16 + 8].set(be @ A[l])
    bp64 = bp64.at[48].set(1.0)
    Wcat = []
    for l in range(3):
        wc = jnp.zeros((H_DIM, 128), F32)
        wc = wc.at[:, 0:96].set(gats[l]["lin"])
        wc = wc.at[:, 96:104].set(S[l])
        wc = wc.at[:, 112:120].set(D[l])
        Wcat.append(wc)
    zero128 = jnp.zeros((128,), F32)

    xp = jnp.zeros((NP, 9), F32).at[:N].set(x)
    eap = jnp.zeros((EP, 3), F32).at[:E].set(edge_attr)
    padv = jnp.full((EP - E,), N, I32)
    loopi = jnp.arange(NP, dtype=I32)
    tailv = jnp.full((EL - EP - NP,), N, I32)
    srcf = jnp.concatenate([src, padv, loopi, tailv]).reshape(EL // 128, 128)
    dstf = jnp.concatenate([dst, padv, loopi, tailv]).reshape(EL // 128, 128)
    bidx = jnp.concatenate([batch.astype(I32), jnp.full((NP - N,), B, I32)])
    zeros16 = jnp.zeros((NP, 16), F32)
    zeros64 = jnp.zeros((NP, 64), F32)
    zeros96 = jnp.zeros((NP, H_DIM), F32)
    hmap = (jnp.arange(96, dtype=I32) // HC).reshape(6, 16)

    (h0,) = _mm(xp, params["node_emb"]["W"], params["node_emb"]["b"], act=False)
    ae64, aemaxb = _mm(eap, Wp64, bp64, act=False, colmax=True)
    aemax = jnp.max(aemaxb, axis=(0, 1))

    degacc = _sc_scatter_deg(dstf, ae64, zeros64)
    loop64, lmaxb = _tc_degcomb(degacc)
    lmax = jnp.max(lmaxb, axis=(0, 1))

    zpad = jnp.zeros((EL - EP - NP, 16), F32)
    hcur = h0
    residual = h0
    hidden = [h0]
    alphas = []
    for l in range(3):
        zs, zmaxb = _mm(hcur, Wcat[l], zero128, act=False, colmax=True)
        zmax = jnp.max(zmaxb, axis=(0, 1))
        xs = zs[:, 0:96]
        asrc = zs[:, 96:112]
        adst = zs[:, 112:128]
        m_ae = jnp.maximum(aemax[l * 16:l * 16 + 8], lmax[l * 16:l * 16 + 8])
        m8 = zmax[96:104] + zmax[112:120] + m_ae
        m8 = jnp.where(m8 >= 0.0, m8, m8 * 0.2)
        m16 = jnp.concatenate([m8, jnp.zeros((8,), F32)])
        aef = jnp.concatenate(
            [ae64[:, l * 16:(l + 1) * 16], loop64[:, l * 16:(l + 1) * 16], zpad])
        ehbm, sparts = _sc_pass1(srcf, dstf, asrc, adst, aef, m16, zeros16)
        sc = _tc_scomb(sparts)
        ahbm, mparts = _sc_pass2(srcf, dstf, ehbm, sc, xs, hmap, zeros96)
        a2 = ahbm.reshape(EL, 16)
        alphas.append(jnp.concatenate([a2[0:E, 0:8], a2[EP:EP + N, 0:8]]))
        g = gats[l]
        bn_scale = g["bn_g"] / jnp.sqrt(g["bn_rv"] + 1e-5)
        bn_shift = g["bn_b"] - g["bn_rm"] * bn_scale
        res = residual if l == 2 else None
        hcur = _tc_post(mparts, g["bias"], bn_scale, bn_shift, res)
        hidden.append(hcur)

    (g1,) = _mm(hidden[3], params["gate1"]["W"], params["gate1"]["b"], act=True)
    Wg2 = jnp.zeros((48, 16), F32).at[:, 0].set(params["gate2"]["W"][:, 0])
    bg2 = jnp.zeros((16,), F32).at[0].set(params["gate2"]["b"][0])
    gv, gmaxb = _mm(g1, Wg2, bg2, act=False, colmax=True)
    mg = jnp.max(gmaxb, axis=(0, 1))[0].reshape(1, 1)

    poolA, poolB = _tc_pool(hidden[0], hidden[1], hidden[2], hidden[3],
                            gv, mg, bidx.reshape(NP, 1))
    maxparts = _sc_maxpool(hidden[3], bidx, zeros96).reshape(NW * B, H_DIM)

    Wpj = jnp.zeros((384, 512), F32)
    bpj = jnp.zeros((512,), F32)
    for i in range(4):
        Wpj = Wpj.at[i * 96:(i + 1) * 96, i * 128:(i + 1) * 128].set(
            params["proj"][i]["W"])
        bpj = bpj.at[i * 128:(i + 1) * 128].set(params["proj"][i]["b"])
    mlps = []
    for t in TASKS:
        mlps.append(params["head_" + t])
        mlps.extend(params["conf_" + t])
    W1 = jnp.concatenate([m[0]["W"] for m in mlps], axis=1)
    b1 = jnp.concatenate([m[0]["b"] for m in mlps])
    W2 = jnp.zeros((576, 288), F32)
    b2 = jnp.concatenate([m[1]["b"] for m in mlps])
    W3 = jnp.zeros((288, 16), F32)
    b3 = jnp.zeros((16,), F32)
    for i, m in enumerate(mlps):
        W2 = W2.at[i * 48:(i + 1) * 48, i * 24:(i + 1) * 24].set(m[1]["W"])
        W3 = W3.at[i * 24:(i + 1) * 24, i].set(m[2]["W"][:, 0])
        b3 = b3.at[i].set(m[2]["b"][0])
    mha = params["mha"]

    outa, shared, pj = _tc_readout(
        poolA, poolB, maxparts,
        params["mlp1"]["W"], params["mlp1"]["b"],
        params["mlp2"]["W"], params["mlp2"]["b"],
        params["sf1"]["W"], params["sf1"]["b"],
        params["sf2"]["W"], params["sf2"]["b"],
        Wpj, bpj, mha["Wv"], mha["bv"], mha["Wo"], mha["bo"],
        W1, b1, W2, b2, W3, b3)

    preds = [outa[:, t:t + 1] for t in range(3)]
    confs = [outa[:, 3 + 2 * t:4 + 2 * t] for t in range(3)]
    uncs = [outa[:, 4 + 2 * t:5 + 2 * t] for t in range(3)]
    proj = [pj[:, i * 128:(i + 1) * 128] for i in range(4)]
    return (*preds, *proj, *confs, shared, *alphas, *uncs)


# speedup vs baseline: 26.6978x; 1.0496x over previous
import functools
import jax
import jax.numpy as jnp
from jax import lax
from jax.experimental import pallas as pl
from jax.experimental.pallas import tpu as pltpu, tpu_sc as plsc

N = 10000
E = 320000
B = 256
H_DIM = 96
HEADS = 8
HC = 12
NUM_LAYERS = 3
TASKS = ["normal", "mcc26", "mkl1"]

NC, NS, LN = 2, 16, 16
NW = NC * NS
NP = 10240
EP = 327680
EL = 344064
ET1 = EL // NW
GT1 = ET1 // 128
ETD = EP // NW
F32 = jnp.float32
I32 = jnp.int32

_MESH = plsc.VectorSubcoreMesh(core_axis_name="c", subcore_axis_name="s")
_SC_PARAMS = pltpu.CompilerParams(use_tc_tiling_on_sc=False)


_GDN = lax.GatherDimensionNumbers(offset_dims=(), collapsed_slice_dims=(0,),
                                  start_index_map=(0,))


def _vgather(vec, idx):
    return lax.gather(vec, idx[:, None], _GDN, (1,),
                      mode=lax.GatherScatterMode.PROMISE_IN_BOUNDS)


def _wid():
    return lax.axis_index("s") * NC + lax.axis_index("c")


def _sc_scatter_deg(dstf, ae64, zeros64):
    CW = 64
    C, KI = 1024, 8
    NCH = ETD // C
    RT = NP // NS

    @functools.partial(
        pl.kernel,
        out_type=jax.ShapeDtypeStruct((NC, NP, CW), F32),
        mesh=_MESH,
        compiler_params=_SC_PARAMS,
        scratch_types=[
            pltpu.VMEM((KI, 128), I32),
            pltpu.VMEM((C, CW), F32),
            pltpu.VMEM_SHARED((NP, CW), F32),
            pltpu.SemaphoreType.DMA,
        ],
    )
    def k(dst_hbm, ae_hbm, z_hbm, out_hbm, didx, vals, acc, sem):
        cid = lax.axis_index("c")
        sid = lax.axis_index("s")
        w = _wid()
        pltpu.sync_copy(z_hbm.at[pl.ds(sid * RT, RT)], acc.at[pl.ds(sid * RT, RT)])
        plsc.subcore_barrier()
        g0 = w * (ETD // 128)
        e0 = w * ETD
        for ch in range(NCH):
            pltpu.sync_copy(dst_hbm.at[pl.ds(g0 + ch * KI, KI)], didx)
            pltpu.sync_copy(ae_hbm.at[pl.ds(e0 + ch * C, C)], vals)
            ds_ = [pltpu.async_copy(vals.at[pl.ds(j * 128, 128)],
                                    acc.at[didx.at[j]], sem, add=True)
                   for j in range(KI)]
            for d in ds_:
                d.wait()
        plsc.subcore_barrier()
        pltpu.sync_copy(acc.at[pl.ds(sid * RT, RT)],
                        out_hbm.at[cid, pl.ds(sid * RT, RT)])

    return k(dstf, ae64, zeros64)


def _sc_pass1(srcf, dstf, asrc, adst, aef, m16, zeros16):
    C, KI = 896, 7
    NCH = ET1 // C
    RT = NP // NS

    @functools.partial(
        pl.kernel,
        out_type=(jax.ShapeDtypeStruct((EL, 16), F32),
                  jax.ShapeDtypeStruct((NC, NP, 16), F32)),
        mesh=_MESH,
        compiler_params=_SC_PARAMS,
        scratch_types=[
            pltpu.VMEM((KI, 128), I32),
            pltpu.VMEM((KI, 128), I32),
            pltpu.VMEM((C, 16), F32),
            pltpu.VMEM((C, 16), F32),
            pltpu.VMEM((C, 16), F32),
            pltpu.VMEM((C, 16), F32),
            pltpu.VMEM((16,), F32),
            pltpu.VMEM_SHARED((NP, 16), F32),
            pltpu.SemaphoreType.DMA,
            pltpu.SemaphoreType.DMA,
        ],
    )
    def k(src_hbm, dst_hbm, as_hbm, ad_hbm, ae_hbm, m_hbm, z_hbm,
          e_hbm, sout_hbm,
          sidx, didx, g1, g2, aev, ev, mv, acc, sem, sem2):
        cid = lax.axis_index("c")
        sid = lax.axis_index("s")
        w = _wid()
        pltpu.sync_copy(z_hbm.at[pl.ds(sid * RT, RT)], acc.at[pl.ds(sid * RT, RT)])
        pltpu.sync_copy(m_hbm, mv)
        plsc.subcore_barrier()
        lanes = lax.iota(I32, 16)
        mk = lanes < 8
        g0 = w * GT1
        e0 = w * ET1
        for ch in range(NCH):
            pltpu.sync_copy(src_hbm.at[pl.ds(g0 + ch * KI, KI)], sidx)
            pltpu.sync_copy(dst_hbm.at[pl.ds(g0 + ch * KI, KI)], didx)
            pltpu.sync_copy(ae_hbm.at[pl.ds(e0 + ch * C, C)], aev)
            ds_ = [pltpu.async_copy(as_hbm.at[sidx.at[j]],
                                    g1.at[pl.ds(j * 128, 128)], sem)
                   for j in range(KI)]
            ds_ += [pltpu.async_copy(ad_hbm.at[didx.at[j]],
                                     g2.at[pl.ds(j * 128, 128)], sem)
                    for j in range(KI)]
            for d in ds_:
                d.wait()
            mvv = mv[...]

            @plsc.parallel_loop(0, C, 1, unroll=4)
            def row(r):
                z = g1[r, :] + g2[r, :] + aev[r, :]
                z = jnp.where(z >= 0.0, z, z * 0.2)
                e = jnp.exp(z - mvv)
                ev[r, :] = jnp.where(mk, e, 0.0)
            pltpu.sync_copy(ev, e_hbm.at[pl.ds(e0 + ch * C, C)])
            ds_ = [pltpu.async_copy(ev.at[pl.ds(j * 128, 128)],
                                    acc.at[didx.at[j]], sem2, add=True)
                   for j in range(KI)]
            for d in ds_:
                d.wait()
        plsc.subcore_barrier()
        pltpu.sync_copy(acc.at[pl.ds(sid * RT, RT)],
                        sout_hbm.at[cid, pl.ds(sid * RT, RT)])

    return k(srcf, dstf, asrc, adst, aef, m16, zeros16)


def _sc_pass2(srcf, dstf, ehbm, schbm, xs, hmap, zeros96):
    C, KI = 384, 3
    NCH = ET1 // C
    RT = NP // NS

    @functools.partial(
        pl.kernel,
        out_type=(jax.ShapeDtypeStruct((EL * 16,), F32),
                  jax.ShapeDtypeStruct((NC, NP, H_DIM), F32)),
        mesh=_MESH,
        compiler_params=_SC_PARAMS,
        scratch_types=[
            pltpu.VMEM((KI, 128), I32),
            pltpu.VMEM((KI, 128), I32),
            pltpu.VMEM((C, 16), F32),
            pltpu.VMEM((C, 16), F32),
            pltpu.VMEM((C * 16,), F32),
            pltpu.VMEM((C, H_DIM), F32),
            pltpu.VMEM((6, 16), I32),
            pltpu.VMEM_SHARED((NP, H_DIM), F32),
            pltpu.SemaphoreType.DMA,
            pltpu.SemaphoreType.DMA,
        ],
    )
    def k(src_hbm, dst_hbm, e_hbm, s_hbm, xs_hbm, hm_hbm, z_hbm,
          a_hbm, mout_hbm,
          sidx, didx, sv, ev, av, xv, hmv, acc, sem, sem2):
        cid = lax.axis_index("c")
        sid = lax.axis_index("s")
        w = _wid()
        pltpu.sync_copy(z_hbm.at[pl.ds(sid * RT, RT)], acc.at[pl.ds(sid * RT, RT)])
        pltpu.sync_copy(hm_hbm, hmv)
        plsc.subcore_barrier()
        hms = [hmv[j, :] for j in range(6)]
        g0 = w * GT1
        e0 = w * ET1
        for ch in range(NCH):
            pltpu.sync_copy(src_hbm.at[pl.ds(g0 + ch * KI, KI)], sidx)
            pltpu.sync_copy(dst_hbm.at[pl.ds(g0 + ch * KI, KI)], didx)
            pltpu.sync_copy(e_hbm.at[pl.ds(e0 + ch * C, C)], ev)
            ds_ = [pltpu.async_copy(s_hbm.at[didx.at[j]],
                                    sv.at[pl.ds(j * 128, 128)], sem)
                   for j in range(KI)]
            ds_ += [pltpu.async_copy(xs_hbm.at[sidx.at[j]],
                                     xv.at[pl.ds(j * 128, 128)], sem)
                    for j in range(KI)]
            for d in ds_:
                d.wait()

            @plsc.parallel_loop(0, C, 1, unroll=4)
            def mrow(r):
                a_r = ev[r, :] / sv[r, :]
                av[pl.ds(r * 16, 16)] = a_r
                for j in range(6):
                    g = _vgather(a_r, hms[j])
                    xv[r, pl.ds(j * 16, 16)] = xv[r, pl.ds(j * 16, 16)] * g
            pltpu.sync_copy(av, a_hbm.at[pl.ds((e0 + ch * C) * 16, C * 16)])
            ds_ = [pltpu.async_copy(xv.at[pl.ds(j * 128, 128)],
                                    acc.at[didx.at[j]], sem2, add=True)
                   for j in range(KI)]
            for d in ds_:
                d.wait()
        plsc.subcore_barrier()
        pltpu.sync_copy(acc.at[pl.ds(sid * RT, RT)],
                        mout_hbm.at[cid, pl.ds(sid * RT, RT)])

    return k(srcf, dstf, ehbm, schbm, xs, hmap, zeros96)


def _sc_maxpool(h, batch_np, zeros96):
    RT = 400

    @functools.partial(
        pl.kernel,
        out_type=jax.ShapeDtypeStruct((NW, B, H_DIM), F32),
        mesh=_MESH,
        compiler_params=_SC_PARAMS,
        scratch_types=[
            pltpu.VMEM((RT, H_DIM), F32),
            pltpu.VMEM((RT,), I32),
            pltpu.VMEM((B, H_DIM), F32),
        ],
    )
    def k(h_hbm, b_hbm, z_hbm, out_hbm, hv, bv, acc):
        w = _wid()
        pltpu.sync_copy(z_hbm.at[pl.ds(0, B)], acc)

        @pl.when(w < 25)
        def _():
            pltpu.sync_copy(h_hbm.at[pl.ds(w * RT, RT)], hv)
            pltpu.sync_copy(b_hbm.at[pl.ds(w * RT, RT)], bv)

            def grp(g, _):
                bjv = bv[pl.ds(g * 16, 16)]
                for t in range(16):
                    b = bjv[t]
                    r = g * 16 + t
                    for j in range(6):
                        cur = acc[b, pl.ds(j * 16, 16)]
                        acc[b, pl.ds(j * 16, 16)] = jnp.maximum(
                            cur, hv[r, pl.ds(j * 16, 16)])
                return 0

            lax.fori_loop(0, RT // 16, grp, 0)

        pltpu.sync_copy(acc, out_hbm.at[w])

    return k(h, batch_np, zeros96)


def _mm(x, w, b, act=False, colmax=False, br=512):
    R, K = x.shape
    W = w.shape[1]
    nb = R // br

    def body(x_ref, w_ref, b_ref, o_ref, *mx):
        acc = jnp.dot(x_ref[...], w_ref[...], preferred_element_type=F32)
        acc = acc + b_ref[...]
        if act:
            acc = jnp.maximum(acc, 0.0)
        o_ref[...] = acc
        if colmax:
            mx[0][...] = jnp.max(acc, axis=0, keepdims=True)[None]

    outs = [jax.ShapeDtypeStruct((R, W), F32)]
    ospecs = [pl.BlockSpec((br, W), lambda i: (i, 0))]
    if colmax:
        outs.append(jax.ShapeDtypeStruct((nb, 1, W), F32))
        ospecs.append(pl.BlockSpec((1, 1, W), lambda i: (i, 0, 0)))
    res = pl.pallas_call(
        body,
        grid=(nb,),
        in_specs=[pl.BlockSpec((br, K), lambda i: (i, 0)),
                  pl.BlockSpec((K, W), lambda i: (0, 0)),
                  pl.BlockSpec((1, W), lambda i: (0, 0))],
        out_specs=ospecs if colmax else ospecs[0],
        out_shape=outs if colmax else outs[0],
    )(x, w, b.reshape(1, W))
    return res if colmax else (res,)


def _tc_degcomb(parts):
    br = 512
    nb = NP // br

    def body(p_ref, o_ref, mx_ref):
        t = p_ref[0] + p_ref[1]
        degc = jnp.maximum(t[:, 48:49], 1.0)
        o = t / degc
        o_ref[...] = o
        mx_ref[...] = jnp.max(o, axis=0, keepdims=True)[None]

    return pl.pallas_call(
        body,
        grid=(nb,),
        in_specs=[pl.BlockSpec((2, br, 64), lambda i: (0, i, 0))],
        out_specs=[pl.BlockSpec((br, 64), lambda i: (i, 0)),
                   pl.BlockSpec((1, 1, 64), lambda i: (i, 0, 0))],
        out_shape=[jax.ShapeDtypeStruct((NP, 64), F32),
                   jax.ShapeDtypeStruct((nb, 1, 64), F32)],
    )(parts)


def _tc_scomb(parts):
    br = 512
    nb = NP // br

    def body(p_ref, o_ref):
        o_ref[...] = p_ref[0] + p_ref[1] + 1e-16

    return pl.pallas_call(
        body,
        grid=(nb,),
        in_specs=[pl.BlockSpec((2, br, 16), lambda i: (0, i, 0))],
        out_specs=pl.BlockSpec((br, 16), lambda i: (i, 0)),
        out_shape=jax.ShapeDtypeStruct((NP, 16), F32),
    )(parts)


def _tc_post(parts, bias, bn_scale, bn_shift, res):
    br = 512
    nb = NP // br
    with_res = res is not None

    def body(p_ref, b_ref, s_ref, t_ref, *rest):
        if with_res:
            r_ref, o_ref = rest
        else:
            (o_ref,) = rest
        v = (p_ref[0] + p_ref[1] + b_ref[...]) * s_ref[...] + t_ref[...]
        v = jnp.maximum(v, 0.0)
        if with_res:
            v = v + r_ref[...]
        o_ref[...] = v

    in_specs = [pl.BlockSpec((2, br, H_DIM), lambda i: (0, i, 0)),
                pl.BlockSpec((1, H_DIM), lambda i: (0, 0)),
                pl.BlockSpec((1, H_DIM), lambda i: (0, 0)),
                pl.BlockSpec((1, H_DIM), lambda i: (0, 0))]
    args = [parts, bias.reshape(1, H_DIM), bn_scale.reshape(1, H_DIM),
            bn_shift.reshape(1, H_DIM)]
    if with_res:
        in_specs.append(pl.BlockSpec((br, H_DIM), lambda i: (i, 0)))
        args.append(res)
    return pl.pallas_call(
        body,
        grid=(nb,),
        in_specs=in_specs,
        out_specs=pl.BlockSpec((br, H_DIM), lambda i: (i, 0)),
        out_shape=jax.ShapeDtypeStruct((NP, H_DIM), F32),
    )(*args)


def _tc_pool(h0, h1, h2, h3, gv, mg, bidx):
    br = 512
    nb = NP // br

    def body(h0r, h1r, h2r, h3r, gr, mgr, br_, oa, ob):
        i = pl.program_id(0)
        oh = (br_[...] == lax.broadcasted_iota(I32, (br, B), 1)).astype(F32)
        e = jnp.exp(gr[...] - mgr[0, 0])
        ec = e[:, 0:1]
        vals = jnp.concatenate([h0r[...], h1r[...], h2r[...], h3r[...],
                                ec * h3r[...]], axis=1)
        pa = lax.dot_general(oh, vals, (((0,), (0,)), ((), ())),
                             preferred_element_type=F32)
        vals2 = jnp.concatenate([e, jnp.ones((br, 16), F32)], axis=1)
        pb = lax.dot_general(oh, vals2, (((0,), (0,)), ((), ())),
                             preferred_element_type=F32)

        @pl.when(i == 0)
        def _():
            oa[...] = pa
            ob[...] = pb

        @pl.when(i > 0)
        def _():
            oa[...] += pa
            ob[...] += pb

    return pl.pallas_call(
        body,
        grid=(nb,),
        in_specs=[pl.BlockSpec((br, H_DIM), lambda i: (i, 0)),
                  pl.BlockSpec((br, H_DIM), lambda i: (i, 0)),
                  pl.BlockSpec((br, H_DIM), lambda i: (i, 0)),
                  pl.BlockSpec((br, H_DIM), lambda i: (i, 0)),
                  pl.BlockSpec((br, 16), lambda i: (i, 0)),
                  pl.BlockSpec((1, 1), lambda i: (0, 0)),
                  pl.BlockSpec((br, 1), lambda i: (i, 0))],
        out_specs=[pl.BlockSpec((B, 480), lambda i: (0, 0)),
                   pl.BlockSpec((B, 32), lambda i: (0, 0))],
        out_shape=[jax.ShapeDtypeStruct((B, 480), F32),
                   jax.ShapeDtypeStruct((B, 32), F32)],
    )(h0, h1, h2, h3, gv, mg, bidx)


def _tc_readout(poolA, poolB, maxparts, Wm1, bm1, Wm2, bm2, Ws1, bs1, Ws2, bs2,
                Wpj, bpj, Wv, bv, Wo, bo, W1, b1, W2, b2, W3, b3):
    def body(pa, pb, mp, wm1, cm1, wm2, cm2, ws1, cs1, ws2, cs2,
             wpj, cpj, wv, cv, wo, co, w1, c1, w2, c2, w3, c3,
             outa, outs, outp):
        cnt = jnp.maximum(pb[:, 16:17], 1.0)
        sg = pb[:, 0:1] + 1e-16
        xm = mp[pl.ds(0, B), :]
        for kk in range(1, NW):
            xm = jnp.maximum(xm, mp[pl.ds(kk * B, B), :])
        x_mean = pa[:, 288:384] / cnt
        x_att = pa[:, 384:480] / sg
        gr = jnp.concatenate([x_mean, xm, x_att], axis=1)
        gr = jnp.maximum(jnp.dot(gr, wm1[...], preferred_element_type=F32) + cm1[...], 0.0)
        gr = jnp.maximum(jnp.dot(gr, wm2[...], preferred_element_type=F32) + cm2[...], 0.0)
        sh = jnp.maximum(jnp.dot(gr, ws1[...], preferred_element_type=F32) + cs1[...], 0.0)
        sh = jnp.maximum(jnp.dot(sh, ws2[...], preferred_element_type=F32) + cs2[...], 0.0)
        pools = pa[:, 0:384] / cnt
        pj = jnp.maximum(jnp.dot(pools, wpj[...], preferred_element_type=F32) + cpj[...], 0.0)
        f = jnp.dot(sh, wv[...], preferred_element_type=F32) + cv[...]
        f = jnp.dot(f, wo[...], preferred_element_type=F32) + co[...]
        h1 = jnp.maximum(jnp.dot(f, w1[...], preferred_element_type=F32) + c1[...], 0.0)
        h2 = jnp.maximum(jnp.dot(h1, w2[...], preferred_element_type=F32) + c2[...], 0.0)
        zl = jnp.dot(h2, w3[...], preferred_element_type=F32) + c3[...]
        z = 1.0 / (1.0 + jnp.exp(-zl))
        cols = []
        for t in range(3):
            cols.append(z[:, 4 * t:4 * t + 1])
        for t in range(3):
            a = z[:, 4 * t + 1:4 * t + 2]
            bb = z[:, 4 * t + 2:4 * t + 3]
            c = z[:, 4 * t + 3:4 * t + 4]
            m = (a + bb + c) / 3.0
            var = ((a - m) ** 2 + (bb - m) ** 2 + (c - m) ** 2) / 2.0
            unc = jnp.sqrt(var)
            cols.append(m * (1.0 - unc * 0.5))
            cols.append(unc)
        outa[...] = jnp.concatenate(cols + [jnp.zeros((B, 7), F32)], axis=1)
        outs[...] = sh
        outp[...] = pj

    full = lambda shp: pl.BlockSpec(shp, lambda: tuple(0 for _ in shp))
    args = [poolA, poolB, maxparts,
            Wm1, bm1.reshape(1, -1), Wm2, bm2.reshape(1, -1),
            Ws1, bs1.reshape(1, -1), Ws2, bs2.reshape(1, -1),
            Wpj, bpj.reshape(1, -1), Wv, bv.reshape(1, -1),
            Wo, bo.reshape(1, -1), W1, b1.reshape(1, -1),
            W2, b2.reshape(1, -1), W3, b3.reshape(1, -1)]
    return pl.pallas_call(
        body,
        in_specs=[full(a.shape) for a in args],
        out_specs=[full((B, 16)), full((B, H_DIM)), full((B, 512))],
        out_shape=[jax.ShapeDtypeStruct((B, 16), F32),
                   jax.ShapeDtypeStruct((B, H_DIM), F32),
                   jax.ShapeDtypeStruct((B, 512), F32)],
    )(*args)


def _fold_att(lin, att):
    return jnp.einsum("dhc,hc->dh", lin.reshape(H_DIM, HEADS, HC), att)


def kernel(x, edge_index, edge_attr, batch, params, return_hidden):
    src = edge_index[0].astype(I32)
    dst = edge_index[1].astype(I32)

    gats = params["gat"]
    A = [_fold_att(g["lin_edge"], g["att_edge"]) for g in gats]
    S = [_fold_att(g["lin"], g["att_src"]) for g in gats]
    D = [_fold_att(g["lin"], g["att_dst"]) for g in gats]
    We, be = params["edge_emb"]["W"], params["edge_emb"]["b"]
    Wp64 = jnp.zeros((3, 64), F32)
    bp64 = jnp.zeros((64,), F32)
    for l in range(3):
        Wp64 = Wp64.at[:, l * 16:l * 16 + 8].set(We @ A[l])
        bp64 = bp64.at[l * 16:l * 16 + 8].set(be @ A[l])
    bp64 = bp64.at[48].set(1.0)
    Wcat = []
    for l in range(3):
        wc = jnp.zeros((H_DIM, 128), F32)
        wc = wc.at[:, 0:96].set(gats[l]["lin"])
        wc = wc.at[:, 96:104].set(S[l])
        wc = wc.at[:, 112:120].set(D[l])
        Wcat.append(wc)
    zero128 = jnp.zeros((128,), F32)

    xp = jnp.zeros((NP, 9), F32).at[:N].set(x)
    eap = jnp.zeros((EP, 3), F32).at[:E].set(edge_attr)
    padv = jnp.full((EP - E,), N, I32)
    loopi = jnp.arange(NP, dtype=I32)
    tailv = jnp.full((EL - EP - NP,), N, I32)
    srcf = jnp.concatenate([src, padv, loopi, tailv]).reshape(EL // 128, 128)
    dstf = jnp.concatenate([dst, padv, loopi, tailv]).reshape(EL // 128, 128)
    bidx = jnp.concatenate([batch.astype(I32), jnp.full((NP - N,), B, I32)])
    zeros16 = jnp.zeros((NP, 16), F32)
    zeros64 = jnp.zeros((NP, 64), F32)
    zeros96 = jnp.zeros((NP, H_DIM), F32)
    hmap = (jnp.arange(96, dtype=I32) // HC).reshape(6, 16)

    (h0,) = _mm(xp, params["node_emb"]["W"], params["node_emb"]["b"], act=False)
    ae64, aemaxb = _mm(eap, Wp64, bp64, act=False, colmax=True)
    aemax = jnp.max(aemaxb, axis=(0, 1))

    degacc = _sc_scatter_deg(dstf, ae64, zeros64)
    loop64, lmaxb = _tc_degcomb(degacc)
    lmax = jnp.max(lmaxb, axis=(0, 1))

    zpad = jnp.zeros((EL - EP - NP, 16), F32)
    hcur = h0
    residual = h0
    hidden = [h0]
    alphas = []
    for l in range(3):
        zs, zmaxb = _mm(hcur, Wcat[l], zero128, act=False, colmax=True)
        zmax = jnp.max(zmaxb, axis=(0, 1))
        xs = zs[:, 0:96]
        asrc = zs[:, 96:112]
        adst = zs[:, 112:128]
        m_ae = jnp.maximum(aemax[l * 16:l * 16 + 8], lmax[l * 16:l * 16 + 8])
        m8 = zmax[96:104] + zmax[112:120] + m_ae
        m8 = jnp.where(m8 >= 0.0, m8, m8 * 0.2)
        m16 = jnp.concatenate([m8, jnp.zeros((8,), F32)])
        aef = jnp.concatenate(
            [ae64[:, l * 16:(l + 1) * 16], loop64[:, l * 16:(l + 1) * 16], zpad])
        ehbm, sparts = _sc_pass1(srcf, dstf, asrc, adst, aef, m16, zeros16)
        sc = _tc_scomb(sparts)
        ahbm, mparts = _sc_pass2(srcf, dstf, ehbm, sc, xs, hmap, zeros96)
        a2 = ahbm.reshape(EL, 16)
        alphas.append(jnp.concatenate([a2[0:E, 0:8], a2[EP:EP + N, 0:8]]))
        g = gats[l]
        bn_scale = g["bn_g"] / jnp.sqrt(g["bn_rv"] + 1e-5)
        bn_shift = g["bn_b"] - g["bn_rm"] * bn_scale
        res = residual if l == 2 else None
        hcur = _tc_post(mparts, g["bias"], bn_scale, bn_shift, res)
        hidden.append(hcur)

    (g1,) = _mm(hidden[3], params["gate1"]["W"], params["gate1"]["b"], act=True)
    Wg2 = jnp.zeros((48, 16), F32).at[:, 0].set(params["gate2"]["W"][:, 0])
    bg2 = jnp.zeros((16,), F32).at[0].set(params["gate2"]["b"][0])
    gv, gmaxb = _mm(g1, Wg2, bg2, act=False, colmax=True)
    mg = jnp.max(gmaxb, axis=(0, 1))[0].reshape(1, 1)

    poolA, poolB = _tc_pool(hidden[0], hidden[1], hidden[2], hidden[3],
                            gv, mg, bidx.reshape(NP, 1))
    maxparts = _sc_maxpool(hidden[3], bidx, zeros96).reshape(NW * B, H_DIM)

    Wpj = jnp.zeros((384, 512), F32)
    bpj = jnp.zeros((512,), F32)
    for i in range(4):
        Wpj = Wpj.at[i * 96:(i + 1) * 96, i * 128:(i + 1) * 128].set(
            params["proj"][i]["W"])
        bpj = bpj.at[i * 128:(i + 1) * 128].set(params["proj"][i]["b"])
    mlps = []
    for t in TASKS:
        mlps.append(params["head_" + t])
        mlps.extend(params["conf_" + t])
    W1 = jnp.concatenate([m[0]["W"] for m in mlps], axis=1)
    b1 = jnp.concatenate([m[0]["b"] for m in mlps])
    W2 = jnp.zeros((576, 288), F32)
    b2 = jnp.concatenate([m[1]["b"] for m in mlps])
    W3 = jnp.zeros((288, 16), F32)
    b3 = jnp.zeros((16,), F32)
    for i, m in enumerate(mlps):
        W2 = W2.at[i * 48:(i + 1) * 48, i * 24:(i + 1) * 24].set(m[1]["W"])
        W3 = W3.at[i * 24:(i + 1) * 24, i].set(m[2]["W"][:, 0])
        b3 = b3.at[i].set(m[2]["b"][0])
    mha = params["mha"]

    outa, shared, pj = _tc_readout(
        poolA, poolB, maxparts,
        params["mlp1"]["W"], params["mlp1"]["b"],
        params["mlp2"]["W"], params["mlp2"]["b"],
        params["sf1"]["W"], params["sf1"]["b"],
        params["sf2"]["W"], params["sf2"]["b"],
        Wpj, bpj, mha["Wv"], mha["bv"], mha["Wo"], mha["bo"],
        W1, b1, W2, b2, W3, b3)

    preds = [outa[:, t:t + 1] for t in range(3)]
    confs = [outa[:, 3 + 2 * t:4 + 2 * t] for t in range(3)]
    uncs = [outa[:, 4 + 2 * t:5 + 2 * t] for t in range(3)]
    proj = [pj[:, i * 128:(i + 1) * 128] for i in range(4)]
    return (*preds, *proj, *confs, shared, *alphas, *uncs)


# speedup vs baseline: 26.7170x; 1.0007x over previous
import functools
import jax
import jax.numpy as jnp
from jax import lax
from jax.experimental import pallas as pl
from jax.experimental.pallas import tpu as pltpu, tpu_sc as plsc

N = 10000
E = 320000
B = 256
H_DIM = 96
HEADS = 8
HC = 12
NUM_LAYERS = 3
TASKS = ["normal", "mcc26", "mkl1"]

NC, NS, LN = 2, 16, 16
NW = NC * NS
NP = 10240
EP = 327680
EL = 344064
ET1 = EL // NW
GT1 = ET1 // 128
ETD = EP // NW
F32 = jnp.float32
I32 = jnp.int32

_MESH = plsc.VectorSubcoreMesh(core_axis_name="c", subcore_axis_name="s")
_SC_PARAMS = pltpu.CompilerParams(use_tc_tiling_on_sc=False)


_GDN = lax.GatherDimensionNumbers(offset_dims=(), collapsed_slice_dims=(0,),
                                  start_index_map=(0,))


def _vgather(vec, idx):
    return lax.gather(vec, idx[:, None], _GDN, (1,),
                      mode=lax.GatherScatterMode.PROMISE_IN_BOUNDS)


def _wid():
    return lax.axis_index("s") * NC + lax.axis_index("c")


def _sc_scatter_deg(dstf, ae64, zeros64):
    CW = 64
    C, KI = 1024, 8
    NCH = ETD // C
    RT = NP // NS

    @functools.partial(
        pl.kernel,
        out_type=jax.ShapeDtypeStruct((NC, NP, CW), F32),
        mesh=_MESH,
        compiler_params=_SC_PARAMS,
        scratch_types=[
            pltpu.VMEM((KI, 128), I32),
            pltpu.VMEM((C, CW), F32),
            pltpu.VMEM_SHARED((NP, CW), F32),
            pltpu.SemaphoreType.DMA,
        ],
    )
    def k(dst_hbm, ae_hbm, z_hbm, out_hbm, didx, vals, acc, sem):
        cid = lax.axis_index("c")
        sid = lax.axis_index("s")
        w = _wid()
        pltpu.sync_copy(z_hbm.at[pl.ds(sid * RT, RT)], acc.at[pl.ds(sid * RT, RT)])
        plsc.subcore_barrier()
        g0 = w * (ETD // 128)
        e0 = w * ETD
        for ch in range(NCH):
            pltpu.sync_copy(dst_hbm.at[pl.ds(g0 + ch * KI, KI)], didx)
            pltpu.sync_copy(ae_hbm.at[pl.ds(e0 + ch * C, C)], vals)
            ds_ = [pltpu.async_copy(vals.at[pl.ds(j * 128, 128)],
                                    acc.at[didx.at[j]], sem, add=True)
                   for j in range(KI)]
            for d in ds_:
                d.wait()
        plsc.subcore_barrier()
        pltpu.sync_copy(acc.at[pl.ds(sid * RT, RT)],
                        out_hbm.at[cid, pl.ds(sid * RT, RT)])

    return k(dstf, ae64, zeros64)


def _sc_pass1(srcf, dstf, asrc, adst, aef, m16, zeros16):
    C, KI = 896, 7
    NCH = ET1 // C
    RT = NP // NS

    @functools.partial(
        pl.kernel,
        out_type=(jax.ShapeDtypeStruct((EL, 16), F32),
                  jax.ShapeDtypeStruct((NC, NP, 16), F32)),
        mesh=_MESH,
        compiler_params=_SC_PARAMS,
        scratch_types=[
            pltpu.VMEM((KI, 128), I32),
            pltpu.VMEM((KI, 128), I32),
            pltpu.VMEM((C, 16), F32),
            pltpu.VMEM((C, 16), F32),
            pltpu.VMEM((C, 16), F32),
            pltpu.VMEM((C, 16), F32),
            pltpu.VMEM((16,), F32),
            pltpu.VMEM_SHARED((NP, 16), F32),
            pltpu.SemaphoreType.DMA,
            pltpu.SemaphoreType.DMA,
        ],
    )
    def k(src_hbm, dst_hbm, as_hbm, ad_hbm, ae_hbm, m_hbm, z_hbm,
          e_hbm, sout_hbm,
          sidx, didx, g1, g2, aev, ev, mv, acc, sem, sem2):
        cid = lax.axis_index("c")
        sid = lax.axis_index("s")
        w = _wid()
        pltpu.sync_copy(z_hbm.at[pl.ds(sid * RT, RT)], acc.at[pl.ds(sid * RT, RT)])
        pltpu.sync_copy(m_hbm, mv)
        plsc.subcore_barrier()
        lanes = lax.iota(I32, 16)
        mk = lanes < 8
        g0 = w * GT1
        e0 = w * ET1
        for ch in range(NCH):
            pltpu.sync_copy(src_hbm.at[pl.ds(g0 + ch * KI, KI)], sidx)
            pltpu.sync_copy(dst_hbm.at[pl.ds(g0 + ch * KI, KI)], didx)
            pltpu.sync_copy(ae_hbm.at[pl.ds(e0 + ch * C, C)], aev)
            ds_ = [pltpu.async_copy(as_hbm.at[sidx.at[j]],
                                    g1.at[pl.ds(j * 128, 128)], sem)
                   for j in range(KI)]
            ds_ += [pltpu.async_copy(ad_hbm.at[didx.at[j]],
                                     g2.at[pl.ds(j * 128, 128)], sem)
                    for j in range(KI)]
            for d in ds_:
                d.wait()
            mvv = mv[...]

            @plsc.parallel_loop(0, C, 1, unroll=4)
            def row(r):
                z = g1[r, :] + g2[r, :] + aev[r, :]
                z = jnp.where(z >= 0.0, z, z * 0.2)
                e = jnp.exp(z - mvv)
                ev[r, :] = jnp.where(mk, e, 0.0)

            pltpu.sync_copy(ev, e_hbm.at[pl.ds(e0 + ch * C, C)])
            ds_ = [pltpu.async_copy(ev.at[pl.ds(j * 128, 128)],
                                    acc.at[didx.at[j]], sem2, add=True)
                   for j in range(KI)]
            for d in ds_:
                d.wait()
        plsc.subcore_barrier()
        pltpu.sync_copy(acc.at[pl.ds(sid * RT, RT)],
                        sout_hbm.at[cid, pl.ds(sid * RT, RT)])

    return k(srcf, dstf, asrc, adst, aef, m16, zeros16)


def _sc_pass2(srcf, dstf, ehbm, schbm, xs, hmap, zeros96):
    C, KI = 384, 3
    NCH = ET1 // C
    RT = NP // NS

    @functools.partial(
        pl.kernel,
        out_type=(jax.ShapeDtypeStruct((EL, 16), F32),
                  jax.ShapeDtypeStruct((NC, NP, H_DIM), F32)),
        mesh=_MESH,
        compiler_params=_SC_PARAMS,
        scratch_types=[
            pltpu.VMEM((KI, 128), I32),
            pltpu.VMEM((KI, 128), I32),
            pltpu.VMEM((C, 16), F32),
            pltpu.VMEM((C, 16), F32),
            pltpu.VMEM((C, H_DIM), F32),
            pltpu.VMEM((6, 16), I32),
            pltpu.VMEM_SHARED((NP, H_DIM), F32),
            pltpu.SemaphoreType.DMA,
            pltpu.SemaphoreType.DMA,
        ],
    )
    def k(src_hbm, dst_hbm, e_hbm, s_hbm, xs_hbm, hm_hbm, z_hbm,
          a_hbm, mout_hbm,
          sidx, didx, sv, ev, xv, hmv, acc, sem, sem2):
        cid = lax.axis_index("c")
        sid = lax.axis_index("s")
        w = _wid()
        pltpu.sync_copy(z_hbm.at[pl.ds(sid * RT, RT)], acc.at[pl.ds(sid * RT, RT)])
        pltpu.sync_copy(hm_hbm, hmv)
        plsc.subcore_barrier()
        hms = [hmv[j, :] for j in range(6)]
        g0 = w * GT1
        e0 = w * ET1
        for ch in range(NCH):
            pltpu.sync_copy(src_hbm.at[pl.ds(g0 + ch * KI, KI)], sidx)
            pltpu.sync_copy(dst_hbm.at[pl.ds(g0 + ch * KI, KI)], didx)
            pltpu.sync_copy(e_hbm.at[pl.ds(e0 + ch * C, C)], ev)
            ds_ = [pltpu.async_copy(s_hbm.at[didx.at[j]],
                                    sv.at[pl.ds(j * 128, 128)], sem)
                   for j in range(KI)]
            ds_ += [pltpu.async_copy(xs_hbm.at[sidx.at[j]],
                                     xv.at[pl.ds(j * 128, 128)], sem)
                    for j in range(KI)]
            for d in ds_:
                d.wait()

            @plsc.parallel_loop(0, C, 1, unroll=4)
            def mrow(r):
                a_r = ev[r, :] / sv[r, :]
                ev[r, :] = a_r
                for j in range(6):
                    g = _vgather(a_r, hms[j])
                    xv[r, pl.ds(j * 16, 16)] = xv[r, pl.ds(j * 16, 16)] * g

            pltpu.sync_copy(ev, a_hbm.at[pl.ds(e0 + ch * C, C)])
            ds_ = [pltpu.async_copy(xv.at[pl.ds(j * 128, 128)],
                                    acc.at[didx.at[j]], sem2, add=True)
                   for j in range(KI)]
            for d in ds_:
                d.wait()
        plsc.subcore_barrier()
        pltpu.sync_copy(acc.at[pl.ds(sid * RT, RT)],
                        mout_hbm.at[cid, pl.ds(sid * RT, RT)])

    return k(srcf, dstf, ehbm, schbm, xs, hmap, zeros96)


def _sc_maxpool(h, batch_np, zeros96):
    RT = 400

    @functools.partial(
        pl.kernel,
        out_type=jax.ShapeDtypeStruct((NW, B, H_DIM), F32),
        mesh=_MESH,
        compiler_params=_SC_PARAMS,
        scratch_types=[
            pltpu.VMEM((RT, H_DIM), F32),
            pltpu.VMEM((RT,), I32),
            pltpu.VMEM((B, H_DIM), F32),
        ],
    )
    def k(h_hbm, b_hbm, z_hbm, out_hbm, hv, bv, acc):
        w = _wid()
        pltpu.sync_copy(z_hbm.at[pl.ds(0, B)], acc)

        @pl.when(w < 25)
        def _():
            pltpu.sync_copy(h_hbm.at[pl.ds(w * RT, RT)], hv)
            pltpu.sync_copy(b_hbm.at[pl.ds(w * RT, RT)], bv)

            def grp(g, _):
                bjv = bv[pl.ds(g * 16, 16)]
                for t in range(16):
                    b = bjv[t]
                    r = g * 16 + t
                    for j in range(6):
                        cur = acc[b, pl.ds(j * 16, 16)]
                        acc[b, pl.ds(j * 16, 16)] = jnp.maximum(
                            cur, hv[r, pl.ds(j * 16, 16)])
                return 0

            lax.fori_loop(0, RT // 16, grp, 0)

        pltpu.sync_copy(acc, out_hbm.at[w])

    return k(h, batch_np, zeros96)


def _mm(x, w, b, act=False, colmax=False, br=512):
    R, K = x.shape
    W = w.shape[1]
    nb = R // br

    def body(x_ref, w_ref, b_ref, o_ref, *mx):
        acc = jnp.dot(x_ref[...], w_ref[...], preferred_element_type=F32)
        acc = acc + b_ref[...]
        if act:
            acc = jnp.maximum(acc, 0.0)
        o_ref[...] = acc
        if colmax:
            mx[0][...] = jnp.max(acc, axis=0, keepdims=True)[None]

    outs = [jax.ShapeDtypeStruct((R, W), F32)]
    ospecs = [pl.BlockSpec((br, W), lambda i: (i, 0))]
    if colmax:
        outs.append(jax.ShapeDtypeStruct((nb, 1, W), F32))
        ospecs.append(pl.BlockSpec((1, 1, W), lambda i: (i, 0, 0)))
    res = pl.pallas_call(
        body,
        grid=(nb,),
        in_specs=[pl.BlockSpec((br, K), lambda i: (i, 0)),
                  pl.BlockSpec((K, W), lambda i: (0, 0)),
                  pl.BlockSpec((1, W), lambda i: (0, 0))],
        out_specs=ospecs if colmax else ospecs[0],
        out_shape=outs if colmax else outs[0],
    )(x, w, b.reshape(1, W))
    return res if colmax else (res,)


def _tc_degcomb(parts):
    br = 512
    nb = NP // br

    def body(p_ref, o_ref, mx_ref):
        t = p_ref[0] + p_ref[1]
        degc = jnp.maximum(t[:, 48:49], 1.0)
        o = t / degc
        o_ref[...] = o
        mx_ref[...] = jnp.max(o, axis=0, keepdims=True)[None]

    return pl.pallas_call(
        body,
        grid=(nb,),
        in_specs=[pl.BlockSpec((2, br, 64), lambda i: (0, i, 0))],
        out_specs=[pl.BlockSpec((br, 64), lambda i: (i, 0)),
                   pl.BlockSpec((1, 1, 64), lambda i: (i, 0, 0))],
        out_shape=[jax.ShapeDtypeStruct((NP, 64), F32),
                   jax.ShapeDtypeStruct((nb, 1, 64), F32)],
    )(parts)


def _tc_scomb(parts):
    br = 512
    nb = NP // br

    def body(p_ref, o_ref):
        o_ref[...] = p_ref[0] + p_ref[1] + 1e-16

    return pl.pallas_call(
        body,
        grid=(nb,),
        in_specs=[pl.BlockSpec((2, br, 16), lambda i: (0, i, 0))],
        out_specs=pl.BlockSpec((br, 16), lambda i: (i, 0)),
        out_shape=jax.ShapeDtypeStruct((NP, 16), F32),
    )(parts)


def _tc_post(parts, bias, bn_scale, bn_shift, res):
    br = 512
    nb = NP // br
    with_res = res is not None

    def body(p_ref, b_ref, s_ref, t_ref, *rest):
        if with_res:
            r_ref, o_ref = rest
        else:
            (o_ref,) = rest
        v = (p_ref[0] + p_ref[1] + b_ref[...]) * s_ref[...] + t_ref[...]
        v = jnp.maximum(v, 0.0)
        if with_res:
            v = v + r_ref[...]
        o_ref[...] = v

    in_specs = [pl.BlockSpec((2, br, H_DIM), lambda i: (0, i, 0)),
                pl.BlockSpec((1, H_DIM), lambda i: (0, 0)),
                pl.BlockSpec((1, H_DIM), lambda i: (0, 0)),
                pl.BlockSpec((1, H_DIM), lambda i: (0, 0))]
    args = [parts, bias.reshape(1, H_DIM), bn_scale.reshape(1, H_DIM),
            bn_shift.reshape(1, H_DIM)]
    if with_res:
        in_specs.append(pl.BlockSpec((br, H_DIM), lambda i: (i, 0)))
        args.append(res)
    return pl.pallas_call(
        body,
        grid=(nb,),
        in_specs=in_specs,
        out_specs=pl.BlockSpec((br, H_DIM), lambda i: (i, 0)),
        out_shape=jax.ShapeDtypeStruct((NP, H_DIM), F32),
    )(*args)


def _tc_pool(h0, h1, h2, h3, gv, mg, bidx):
    br = 512
    nb = NP // br

    def body(h0r, h1r, h2r, h3r, gr, mgr, br_, oa, ob):
        i = pl.program_id(0)
        oh = (br_[...] == lax.broadcasted_iota(I32, (br, B), 1)).astype(F32)
        e = jnp.exp(gr[...] - mgr[0, 0])
        ec = e[:, 0:1]
        vals = jnp.concatenate([h0r[...], h1r[...], h2r[...], h3r[...],
                                ec * h3r[...]], axis=1)
        pa = lax.dot_general(oh, vals, (((0,), (0,)), ((), ())),
                             preferred_element_type=F32)
        vals2 = jnp.concatenate([e, jnp.ones((br, 16), F32)], axis=1)
        pb = lax.dot_general(oh, vals2, (((0,), (0,)), ((), ())),
                             preferred_element_type=F32)

        @pl.when(i == 0)
        def _():
            oa[...] = pa
            ob[...] = pb

        @pl.when(i > 0)
        def _():
            oa[...] += pa
            ob[...] += pb

    return pl.pallas_call(
        body,
        grid=(nb,),
        in_specs=[pl.BlockSpec((br, H_DIM), lambda i: (i, 0)),
                  pl.BlockSpec((br, H_DIM), lambda i: (i, 0)),
                  pl.BlockSpec((br, H_DIM), lambda i: (i, 0)),
                  pl.BlockSpec((br, H_DIM), lambda i: (i, 0)),
                  pl.BlockSpec((br, 16), lambda i: (i, 0)),
                  pl.BlockSpec((1, 1), lambda i: (0, 0)),
                  pl.BlockSpec((br, 1), lambda i: (i, 0))],
        out_specs=[pl.BlockSpec((B, 480), lambda i: (0, 0)),
                   pl.BlockSpec((B, 32), lambda i: (0, 0))],
        out_shape=[jax.ShapeDtypeStruct((B, 480), F32),
                   jax.ShapeDtypeStruct((B, 32), F32)],
    )(h0, h1, h2, h3, gv, mg, bidx)


def _tc_readout(poolA, poolB, maxparts, Wm1, bm1, Wm2, bm2, Ws1, bs1, Ws2, bs2,
                Wpj, bpj, Wv, bv, Wo, bo, W1, b1, W2, b2, W3, b3):
    def body(pa, pb, mp, wm1, cm1, wm2, cm2, ws1, cs1, ws2, cs2,
             wpj, cpj, wv, cv, wo, co, w1, c1, w2, c2, w3, c3,
             outa, outs, outp):
        cnt = jnp.maximum(pb[:, 16:17], 1.0)
        sg = pb[:, 0:1] + 1e-16
        xm = mp[pl.ds(0, B), :]
        for kk in range(1, NW):
            xm = jnp.maximum(xm, mp[pl.ds(kk * B, B), :])
        x_mean = pa[:, 288:384] / cnt
        x_att = pa[:, 384:480] / sg
        gr = jnp.concatenate([x_mean, xm, x_att], axis=1)
        gr = jnp.maximum(jnp.dot(gr, wm1[...], preferred_element_type=F32) + cm1[...], 0.0)
        gr = jnp.maximum(jnp.dot(gr, wm2[...], preferred_element_type=F32) + cm2[...], 0.0)
        sh = jnp.maximum(jnp.dot(gr, ws1[...], preferred_element_type=F32) + cs1[...], 0.0)
        sh = jnp.maximum(jnp.dot(sh, ws2[...], preferred_element_type=F32) + cs2[...], 0.0)
        pools = pa[:, 0:384] / cnt
        pj = jnp.maximum(jnp.dot(pools, wpj[...], preferred_element_type=F32) + cpj[...], 0.0)
        f = jnp.dot(sh, wv[...], preferred_element_type=F32) + cv[...]
        f = jnp.dot(f, wo[...], preferred_element_type=F32) + co[...]
        h1 = jnp.maximum(jnp.dot(f, w1[...], preferred_element_type=F32) + c1[...], 0.0)
        h2 = jnp.maximum(jnp.dot(h1, w2[...], preferred_element_type=F32) + c2[...], 0.0)
        zl = jnp.dot(h2, w3[...], preferred_element_type=F32) + c3[...]
        z = 1.0 / (1.0 + jnp.exp(-zl))
        cols = []
        for t in range(3):
            cols.append(z[:, 4 * t:4 * t + 1])
        for t in range(3):
            a = z[:, 4 * t + 1:4 * t + 2]
            bb = z[:, 4 * t + 2:4 * t + 3]
            c = z[:, 4 * t + 3:4 * t + 4]
            m = (a + bb + c) / 3.0
            var = ((a - m) ** 2 + (bb - m) ** 2 + (c - m) ** 2) / 2.0
            unc = jnp.sqrt(var)
            cols.append(m * (1.0 - unc * 0.5))
            cols.append(unc)
        outa[...] = jnp.concatenate(cols + [jnp.zeros((B, 7), F32)], axis=1)
        outs[...] = sh
        outp[...] = pj

    full = lambda shp: pl.BlockSpec(shp, lambda: tuple(0 for _ in shp))
    args = [poolA, poolB, maxparts,
            Wm1, bm1.reshape(1, -1), Wm2, bm2.reshape(1, -1),
            Ws1, bs1.reshape(1, -1), Ws2, bs2.reshape(1, -1),
            Wpj, bpj.reshape(1, -1), Wv, bv.reshape(1, -1),
            Wo, bo.reshape(1, -1), W1, b1.reshape(1, -1),
            W2, b2.reshape(1, -1), W3, b3.reshape(1, -1)]
    return pl.pallas_call(
        body,
        in_specs=[full(a.shape) for a in args],
        out_specs=[full((B, 16)), full((B, H_DIM)), full((B, 512))],
        out_shape=[jax.ShapeDtypeStruct((B, 16), F32),
                   jax.ShapeDtypeStruct((B, H_DIM), F32),
                   jax.ShapeDtypeStruct((B, 512), F32)],
    )(*args)


def _fold_att(lin, att):
    return jnp.einsum("dhc,hc->dh", lin.reshape(H_DIM, HEADS, HC), att)


def kernel(x, edge_index, edge_attr, batch, params, return_hidden):
    src = edge_index[0].astype(I32)
    dst = edge_index[1].astype(I32)

    gats = params["gat"]
    A = [_fold_att(g["lin_edge"], g["att_edge"]) for g in gats]
    S = [_fold_att(g["lin"], g["att_src"]) for g in gats]
    D = [_fold_att(g["lin"], g["att_dst"]) for g in gats]
    We, be = params["edge_emb"]["W"], params["edge_emb"]["b"]
    Wp64 = jnp.zeros((3, 64), F32)
    bp64 = jnp.zeros((64,), F32)
    for l in range(3):
        Wp64 = Wp64.at[:, l * 16:l * 16 + 8].set(We @ A[l])
        bp64 = bp64.at[l * 16:l * 16 + 8].set(be @ A[l])
    bp64 = bp64.at[48].set(1.0)
    Wcat = []
    for l in range(3):
        wc = jnp.zeros((H_DIM, 128), F32)
        wc = wc.at[:, 0:96].set(gats[l]["lin"])
        wc = wc.at[:, 96:104].set(S[l])
        wc = wc.at[:, 112:120].set(D[l])
        Wcat.append(wc)
    zero128 = jnp.zeros((128,), F32)

    xp = jnp.zeros((NP, 9), F32).at[:N].set(x)
    eap = jnp.zeros((EP, 3), F32).at[:E].set(edge_attr)
    padv = jnp.full((EP - E,), N, I32)
    loopi = jnp.arange(NP, dtype=I32)
    tailv = jnp.full((EL - EP - NP,), N, I32)
    srcf = jnp.concatenate([src, padv, loopi, tailv]).reshape(EL // 128, 128)
    dstf = jnp.concatenate([dst, padv, loopi, tailv]).reshape(EL // 128, 128)
    bidx = jnp.concatenate([batch.astype(I32), jnp.full((NP - N,), B, I32)])
    zeros16 = jnp.zeros((NP, 16), F32)
    zeros64 = jnp.zeros((NP, 64), F32)
    zeros96 = jnp.zeros((NP, H_DIM), F32)
    hmap = (jnp.arange(96, dtype=I32) // HC).reshape(6, 16)

    (h0,) = _mm(xp, params["node_emb"]["W"], params["node_emb"]["b"], act=False)
    ae64, aemaxb = _mm(eap, Wp64, bp64, act=False, colmax=True)
    aemax = jnp.max(aemaxb, axis=(0, 1))

    degacc = _sc_scatter_deg(dstf, ae64, zeros64)
    loop64, lmaxb = _tc_degcomb(degacc)
    lmax = jnp.max(lmaxb, axis=(0, 1))

    zpad = jnp.zeros((EL - EP - NP, 16), F32)
    hcur = h0
    residual = h0
    hidden = [h0]
    alphas = []
    for l in range(3):
        zs, zmaxb = _mm(hcur, Wcat[l], zero128, act=False, colmax=True)
        zmax = jnp.max(zmaxb, axis=(0, 1))
        xs = zs[:, 0:96]
        asrc = zs[:, 96:112]
        adst = zs[:, 112:128]
        m_ae = jnp.maximum(aemax[l * 16:l * 16 + 8], lmax[l * 16:l * 16 + 8])
        m8 = zmax[96:104] + zmax[112:120] + m_ae
        m8 = jnp.where(m8 >= 0.0, m8, m8 * 0.2)
        m16 = jnp.concatenate([m8, jnp.zeros((8,), F32)])
        aef = jnp.concatenate(
            [ae64[:, l * 16:(l + 1) * 16], loop64[:, l * 16:(l + 1) * 16], zpad])
        ehbm, sparts = _sc_pass1(srcf, dstf, asrc, adst, aef, m16, zeros16)
        sc = _tc_scomb(sparts)
        ahbm, mparts = _sc_pass2(srcf, dstf, ehbm, sc, xs, hmap, zeros96)
        alphas.append(jnp.concatenate([ahbm[0:E, 0:8], ahbm[EP:EP + N, 0:8]]))
        g = gats[l]
        bn_scale = g["bn_g"] / jnp.sqrt(g["bn_rv"] + 1e-5)
        bn_shift = g["bn_b"] - g["bn_rm"] * bn_scale
        res = residual if l == 2 else None
        hcur = _tc_post(mparts, g["bias"], bn_scale, bn_shift, res)
        hidden.append(hcur)

    (g1,) = _mm(hidden[3], params["gate1"]["W"], params["gate1"]["b"], act=True)
    Wg2 = jnp.zeros((48, 16), F32).at[:, 0].set(params["gate2"]["W"][:, 0])
    bg2 = jnp.zeros((16,), F32).at[0].set(params["gate2"]["b"][0])
    gv, gmaxb = _mm(g1, Wg2, bg2, act=False, colmax=True)
    mg = jnp.max(gmaxb, axis=(0, 1))[0].reshape(1, 1)

    poolA, poolB = _tc_pool(hidden[0], hidden[1], hidden[2], hidden[3],
                            gv, mg, bidx.reshape(NP, 1))
    maxparts = _sc_maxpool(hidden[3], bidx, zeros96).reshape(NW * B, H_DIM)

    Wpj = jnp.zeros((384, 512), F32)
    bpj = jnp.zeros((512,), F32)
    for i in range(4):
        Wpj = Wpj.at[i * 96:(i + 1) * 96, i * 128:(i + 1) * 128].set(
            params["proj"][i]["W"])
        bpj = bpj.at[i * 128:(i + 1) * 128].set(params["proj"][i]["b"])
    mlps = []
    for t in TASKS:
        mlps.append(params["head_" + t])
        mlps.extend(params["conf_" + t])
    W1 = jnp.concatenate([m[0]["W"] for m in mlps], axis=1)
    b1 = jnp.concatenate([m[0]["b"] for m in mlps])
    W2 = jnp.zeros((576, 288), F32)
    b2 = jnp.concatenate([m[1]["b"] for m in mlps])
    W3 = jnp.zeros((288, 16), F32)
    b3 = jnp.zeros((16,), F32)
    for i, m in enumerate(mlps):
        W2 = W2.at[i * 48:(i + 1) * 48, i * 24:(i + 1) * 24].set(m[1]["W"])
        W3 = W3.at[i * 24:(i + 1) * 24, i].set(m[2]["W"][:, 0])
        b3 = b3.at[i].set(m[2]["b"][0])
    mha = params["mha"]

    outa, shared, pj = _tc_readout(
        poolA, poolB, maxparts,
        params["mlp1"]["W"], params["mlp1"]["b"],
        params["mlp2"]["W"], params["mlp2"]["b"],
        params["sf1"]["W"], params["sf1"]["b"],
        params["sf2"]["W"], params["sf2"]["b"],
        Wpj, bpj, mha["Wv"], mha["bv"], mha["Wo"], mha["bo"],
        W1, b1, W2, b2, W3, b3)

    preds = [outa[:, t:t + 1] for t in range(3)]
    confs = [outa[:, 3 + 2 * t:4 + 2 * t] for t in range(3)]
    uncs = [outa[:, 4 + 2 * t:5 + 2 * t] for t in range(3)]
    proj = [pj[:, i * 128:(i + 1) * 128] for i in range(4)]
    return (*preds, *proj, *confs, shared, *alphas, *uncs)


# speedup vs baseline: 27.2452x; 1.0198x over previous
import functools
import jax
import jax.numpy as jnp
from jax import lax
from jax.experimental import pallas as pl
from jax.experimental.pallas import tpu as pltpu, tpu_sc as plsc

N = 10000
E = 320000
B = 256
H_DIM = 96
HEADS = 8
HC = 12
NUM_LAYERS = 3
TASKS = ["normal", "mcc26", "mkl1"]

NC, NS, LN = 2, 16, 16
NW = NC * NS
NP = 10240
EP = 327680
EL = 344064
ET1 = EL // NW
GT1 = ET1 // 128
ETD = EP // NW
F32 = jnp.float32
I32 = jnp.int32

_MESH = plsc.VectorSubcoreMesh(core_axis_name="c", subcore_axis_name="s")
_SC_PARAMS = pltpu.CompilerParams(use_tc_tiling_on_sc=False)


_GDN = lax.GatherDimensionNumbers(offset_dims=(), collapsed_slice_dims=(0,),
                                  start_index_map=(0,))


def _vgather(vec, idx):
    return lax.gather(vec, idx[:, None], _GDN, (1,),
                      mode=lax.GatherScatterMode.PROMISE_IN_BOUNDS)


def _wid():
    return lax.axis_index("s") * NC + lax.axis_index("c")


def _sc_scatter_deg(dstf, ae64, zeros64):
    CW = 64
    C, KI = 1024, 8
    NCH = ETD // C
    RT = NP // NS

    @functools.partial(
        pl.kernel,
        out_type=jax.ShapeDtypeStruct((NC, NP, CW), F32),
        mesh=_MESH,
        compiler_params=_SC_PARAMS,
        scratch_types=[
            pltpu.VMEM((KI, 128), I32),
            pltpu.VMEM((C, CW), F32),
            pltpu.VMEM_SHARED((NP, CW), F32),
            pltpu.SemaphoreType.DMA,
        ],
    )
    def k(dst_hbm, ae_hbm, z_hbm, out_hbm, didx, vals, acc, sem):
        cid = lax.axis_index("c")
        sid = lax.axis_index("s")
        w = _wid()
        pltpu.sync_copy(z_hbm.at[pl.ds(sid * RT, RT)], acc.at[pl.ds(sid * RT, RT)])
        plsc.subcore_barrier()
        g0 = w * (ETD // 128)
        e0 = w * ETD
        for ch in range(NCH):
            pltpu.sync_copy(dst_hbm.at[pl.ds(g0 + ch * KI, KI)], didx)
            pltpu.sync_copy(ae_hbm.at[pl.ds(e0 + ch * C, C)], vals)
            ds_ = [pltpu.async_copy(vals.at[pl.ds(j * 128, 128)],
                                    acc.at[didx.at[j]], sem, add=True)
                   for j in range(KI)]
            for d in ds_:
                d.wait()
        plsc.subcore_barrier()
        pltpu.sync_copy(acc.at[pl.ds(sid * RT, RT)],
                        out_hbm.at[cid, pl.ds(sid * RT, RT)])

    return k(dstf, ae64, zeros64)


def _sc_pass1(srcf, dstf, asrc, adst, aef, m16, zeros16):
    C, KI = 896, 7
    NCH = ET1 // C
    RT = NP // NS

    @functools.partial(
        pl.kernel,
        out_type=(jax.ShapeDtypeStruct((EL, 16), F32),
                  jax.ShapeDtypeStruct((NC, NP, 16), F32)),
        mesh=_MESH,
        compiler_params=_SC_PARAMS,
        scratch_types=[
            pltpu.VMEM((KI, 128), I32),
            pltpu.VMEM((KI, 128), I32),
            pltpu.VMEM((C, 16), F32),
            pltpu.VMEM((C, 16), F32),
            pltpu.VMEM((C, 16), F32),
            pltpu.VMEM((C, 16), F32),
            pltpu.VMEM((16,), F32),
            pltpu.VMEM_SHARED((NP, 16), F32),
            pltpu.SemaphoreType.DMA,
            pltpu.SemaphoreType.DMA,
        ],
    )
    def k(src_hbm, dst_hbm, as_hbm, ad_hbm, ae_hbm, m_hbm, z_hbm,
          e_hbm, sout_hbm,
          sidx, didx, g1, g2, aev, ev, mv, acc, sem, sem2):
        cid = lax.axis_index("c")
        sid = lax.axis_index("s")
        w = _wid()
        pltpu.sync_copy(z_hbm.at[pl.ds(sid * RT, RT)], acc.at[pl.ds(sid * RT, RT)])
        pltpu.sync_copy(m_hbm, mv)
        plsc.subcore_barrier()
        lanes = lax.iota(I32, 16)
        mk = lanes < 8
        g0 = w * GT1
        e0 = w * ET1
        for ch in range(NCH):
            ld = [pltpu.async_copy(src_hbm.at[pl.ds(g0 + ch * KI, KI)], sidx, sem),
                  pltpu.async_copy(dst_hbm.at[pl.ds(g0 + ch * KI, KI)], didx, sem),
                  pltpu.async_copy(ae_hbm.at[pl.ds(e0 + ch * C, C)], aev, sem)]
            for d in ld:
                d.wait()
            ds_ = [pltpu.async_copy(as_hbm.at[sidx.at[j]],
                                    g1.at[pl.ds(j * 128, 128)], sem)
                   for j in range(KI)]
            ds_ += [pltpu.async_copy(ad_hbm.at[didx.at[j]],
                                     g2.at[pl.ds(j * 128, 128)], sem)
                    for j in range(KI)]
            for d in ds_:
                d.wait()
            mvv = mv[...]

            @plsc.parallel_loop(0, C, 1, unroll=4)
            def row(r):
                z = g1[r, :] + g2[r, :] + aev[r, :]
                z = jnp.where(z >= 0.0, z, z * 0.2)
                e = jnp.exp(z - mvv)
                ev[r, :] = jnp.where(mk, e, 0.0)

            pltpu.sync_copy(ev, e_hbm.at[pl.ds(e0 + ch * C, C)])
            ds_ = [pltpu.async_copy(ev.at[pl.ds(j * 128, 128)],
                                    acc.at[didx.at[j]], sem2, add=True)
                   for j in range(KI)]
            for d in ds_:
                d.wait()
        plsc.subcore_barrier()
        pltpu.sync_copy(acc.at[pl.ds(sid * RT, RT)],
                        sout_hbm.at[cid, pl.ds(sid * RT, RT)])

    return k(srcf, dstf, asrc, adst, aef, m16, zeros16)


def _sc_pass2(srcf, dstf, ehbm, schbm, xs, hmap, zeros96):
    C, KI = 512, 4
    NCH = ET1 // C
    RT = NP // NS

    @functools.partial(
        pl.kernel,
        out_type=(jax.ShapeDtypeStruct((EL, 16), F32),
                  jax.ShapeDtypeStruct((NC, NP, H_DIM), F32)),
        mesh=_MESH,
        compiler_params=_SC_PARAMS,
        scratch_types=[
            pltpu.VMEM((KI, 128), I32),
            pltpu.VMEM((KI, 128), I32),
            pltpu.VMEM((C, 16), F32),
            pltpu.VMEM((C, 16), F32),
            pltpu.VMEM((C, H_DIM), F32),
            pltpu.VMEM((6, 16), I32),
            pltpu.VMEM_SHARED((NP, H_DIM), F32),
            pltpu.SemaphoreType.DMA,
            pltpu.SemaphoreType.DMA,
        ],
    )
    def k(src_hbm, dst_hbm, e_hbm, s_hbm, xs_hbm, hm_hbm, z_hbm,
          a_hbm, mout_hbm,
          sidx, didx, sv, ev, xv, hmv, acc, sem, sem2):
        cid = lax.axis_index("c")
        sid = lax.axis_index("s")
        w = _wid()
        pltpu.sync_copy(z_hbm.at[pl.ds(sid * RT, RT)], acc.at[pl.ds(sid * RT, RT)])
        pltpu.sync_copy(hm_hbm, hmv)
        plsc.subcore_barrier()
        hms = [hmv[j, :] for j in range(6)]
        g0 = w * GT1
        e0 = w * ET1
        for ch in range(NCH):
            ld = [pltpu.async_copy(src_hbm.at[pl.ds(g0 + ch * KI, KI)], sidx, sem),
                  pltpu.async_copy(dst_hbm.at[pl.ds(g0 + ch * KI, KI)], didx, sem),
                  pltpu.async_copy(e_hbm.at[pl.ds(e0 + ch * C, C)], ev, sem)]
            for d in ld:
                d.wait()
            ds_ = [pltpu.async_copy(s_hbm.at[didx.at[j]],
                                    sv.at[pl.ds(j * 128, 128)], sem)
                   for j in range(KI)]
            ds_ += [pltpu.async_copy(xs_hbm.at[sidx.at[j]],
                                     xv.at[pl.ds(j * 128, 128)], sem)
                    for j in range(KI)]
            for d in ds_:
                d.wait()

            @plsc.parallel_loop(0, C, 1, unroll=4)
            def mrow(r):
                a_r = ev[r, :] / sv[r, :]
                ev[r, :] = a_r
                for j in range(6):
                    g = _vgather(a_r, hms[j])
                    xv[r, pl.ds(j * 16, 16)] = xv[r, pl.ds(j * 16, 16)] * g

            pltpu.sync_copy(ev, a_hbm.at[pl.ds(e0 + ch * C, C)])
            ds_ = [pltpu.async_copy(xv.at[pl.ds(j * 128, 128)],
                                    acc.at[didx.at[j]], sem2, add=True)
                   for j in range(KI)]
            for d in ds_:
                d.wait()
        plsc.subcore_barrier()
        pltpu.sync_copy(acc.at[pl.ds(sid * RT, RT)],
                        mout_hbm.at[cid, pl.ds(sid * RT, RT)])

    return k(srcf, dstf, ehbm, schbm, xs, hmap, zeros96)


def _sc_maxpool(h, batch_np, zeros96):
    RT = 400

    @functools.partial(
        pl.kernel,
        out_type=jax.ShapeDtypeStruct((NW, B, H_DIM), F32),
        mesh=_MESH,
        compiler_params=_SC_PARAMS,
        scratch_types=[
            pltpu.VMEM((RT, H_DIM), F32),
            pltpu.VMEM((RT,), I32),
            pltpu.VMEM((B, H_DIM), F32),
        ],
    )
    def k(h_hbm, b_hbm, z_hbm, out_hbm, hv, bv, acc):
        w = _wid()
        pltpu.sync_copy(z_hbm.at[pl.ds(0, B)], acc)

        @pl.when(w < 25)
        def _():
            pltpu.sync_copy(h_hbm.at[pl.ds(w * RT, RT)], hv)
            pltpu.sync_copy(b_hbm.at[pl.ds(w * RT, RT)], bv)

            def grp(g, _):
                bjv = bv[pl.ds(g * 16, 16)]
                for t in range(16):
                    b = bjv[t]
                    r = g * 16 + t
                    for j in range(6):
                        cur = acc[b, pl.ds(j * 16, 16)]
                        acc[b, pl.ds(j * 16, 16)] = jnp.maximum(
                            cur, hv[r, pl.ds(j * 16, 16)])
                return 0

            lax.fori_loop(0, RT // 16, grp, 0)

        pltpu.sync_copy(acc, out_hbm.at[w])

    return k(h, batch_np, zeros96)


def _mm(x, w, b, act=False, colmax=False, br=512):
    R, K = x.shape
    W = w.shape[1]
    nb = R // br

    def body(x_ref, w_ref, b_ref, o_ref, *mx):
        acc = jnp.dot(x_ref[...], w_ref[...], preferred_element_type=F32)
        acc = acc + b_ref[...]
        if act:
            acc = jnp.maximum(acc, 0.0)
        o_ref[...] = acc
        if colmax:
            mx[0][...] = jnp.max(acc, axis=0, keepdims=True)[None]

    outs = [jax.ShapeDtypeStruct((R, W), F32)]
    ospecs = [pl.BlockSpec((br, W), lambda i: (i, 0))]
    if colmax:
        outs.append(jax.ShapeDtypeStruct((nb, 1, W), F32))
        ospecs.append(pl.BlockSpec((1, 1, W), lambda i: (i, 0, 0)))
    res = pl.pallas_call(
        body,
        grid=(nb,),
        in_specs=[pl.BlockSpec((br, K), lambda i: (i, 0)),
                  pl.BlockSpec((K, W), lambda i: (0, 0)),
                  pl.BlockSpec((1, W), lambda i: (0, 0))],
        out_specs=ospecs if colmax else ospecs[0],
        out_shape=outs if colmax else outs[0],
    )(x, w, b.reshape(1, W))
    return res if colmax else (res,)


def _tc_degcomb(parts):
    br = 512
    nb = NP // br

    def body(p_ref, o_ref, mx_ref):
        t = p_ref[0] + p_ref[1]
        degc = jnp.maximum(t[:, 48:49], 1.0)
        o = t / degc
        o_ref[...] = o
        mx_ref[...] = jnp.max(o, axis=0, keepdims=True)[None]

    return pl.pallas_call(
        body,
        grid=(nb,),
        in_specs=[pl.BlockSpec((2, br, 64), lambda i: (0, i, 0))],
        out_specs=[pl.BlockSpec((br, 64), lambda i: (i, 0)),
                   pl.BlockSpec((1, 1, 64), lambda i: (i, 0, 0))],
        out_shape=[jax.ShapeDtypeStruct((NP, 64), F32),
                   jax.ShapeDtypeStruct((nb, 1, 64), F32)],
    )(parts)


def _tc_scomb(parts):
    br = 512
    nb = NP // br

    def body(p_ref, o_ref):
        o_ref[...] = p_ref[0] + p_ref[1] + 1e-16

    return pl.pallas_call(
        body,
        grid=(nb,),
        in_specs=[pl.BlockSpec((2, br, 16), lambda i: (0, i, 0))],
        out_specs=pl.BlockSpec((br, 16), lambda i: (i, 0)),
        out_shape=jax.ShapeDtypeStruct((NP, 16), F32),
    )(parts)


def _tc_post(parts, bias, bn_scale, bn_shift, res):
    br = 512
    nb = NP // br
    with_res = res is not None

    def body(p_ref, b_ref, s_ref, t_ref, *rest):
        if with_res:
            r_ref, o_ref = rest
        else:
            (o_ref,) = rest
        v = (p_ref[0] + p_ref[1] + b_ref[...]) * s_ref[...] + t_ref[...]
        v = jnp.maximum(v, 0.0)
        if with_res:
            v = v + r_ref[...]
        o_ref[...] = v

    in_specs = [pl.BlockSpec((2, br, H_DIM), lambda i: (0, i, 0)),
                pl.BlockSpec((1, H_DIM), lambda i: (0, 0)),
                pl.BlockSpec((1, H_DIM), lambda i: (0, 0)),
                pl.BlockSpec((1, H_DIM), lambda i: (0, 0))]
    args = [parts, bias.reshape(1, H_DIM), bn_scale.reshape(1, H_DIM),
            bn_shift.reshape(1, H_DIM)]
    if with_res:
        in_specs.append(pl.BlockSpec((br, H_DIM), lambda i: (i, 0)))
        args.append(res)
    return pl.pallas_call(
        body,
        grid=(nb,),
        in_specs=in_specs,
        out_specs=pl.BlockSpec((br, H_DIM), lambda i: (i, 0)),
        out_shape=jax.ShapeDtypeStruct((NP, H_DIM), F32),
    )(*args)


def _tc_pool(h0, h1, h2, h3, gv, mg, bidx):
    br = 512
    nb = NP // br

    def body(h0r, h1r, h2r, h3r, gr, mgr, br_, oa, ob):
        i = pl.program_id(0)
        oh = (br_[...] == lax.broadcasted_iota(I32, (br, B), 1)).astype(F32)
        e = jnp.exp(gr[...] - mgr[0, 0])
        ec = e[:, 0:1]
        vals = jnp.concatenate([h0r[...], h1r[...], h2r[...], h3r[...],
                                ec * h3r[...]], axis=1)
        pa = lax.dot_general(oh, vals, (((0,), (0,)), ((), ())),
                             preferred_element_type=F32)
        vals2 = jnp.concatenate([e, jnp.ones((br, 16), F32)], axis=1)
        pb = lax.dot_general(oh, vals2, (((0,), (0,)), ((), ())),
                             preferred_element_type=F32)

        @pl.when(i == 0)
        def _():
            oa[...] = pa
            ob[...] = pb

        @pl.when(i > 0)
        def _():
            oa[...] += pa
            ob[...] += pb

    return pl.pallas_call(
        body,
        grid=(nb,),
        in_specs=[pl.BlockSpec((br, H_DIM), lambda i: (i, 0)),
                  pl.BlockSpec((br, H_DIM), lambda i: (i, 0)),
                  pl.BlockSpec((br, H_DIM), lambda i: (i, 0)),
                  pl.BlockSpec((br, H_DIM), lambda i: (i, 0)),
                  pl.BlockSpec((br, 16), lambda i: (i, 0)),
                  pl.BlockSpec((1, 1), lambda i: (0, 0)),
                  pl.BlockSpec((br, 1), lambda i: (i, 0))],
        out_specs=[pl.BlockSpec((B, 480), lambda i: (0, 0)),
                   pl.BlockSpec((B, 32), lambda i: (0, 0))],
        out_shape=[jax.ShapeDtypeStruct((B, 480), F32),
                   jax.ShapeDtypeStruct((B, 32), F32)],
    )(h0, h1, h2, h3, gv, mg, bidx)


def _tc_readout(poolA, poolB, maxparts, Wm1, bm1, Wm2, bm2, Ws1, bs1, Ws2, bs2,
                Wpj, bpj, Wv, bv, Wo, bo, W1, b1, W2, b2, W3, b3):
    def body(pa, pb, mp, wm1, cm1, wm2, cm2, ws1, cs1, ws2, cs2,
             wpj, cpj, wv, cv, wo, co, w1, c1, w2, c2, w3, c3,
             outa, outs, outp):
        cnt = jnp.maximum(pb[:, 16:17], 1.0)
        sg = pb[:, 0:1] + 1e-16
        xm = mp[pl.ds(0, B), :]
        for kk in range(1, NW):
            xm = jnp.maximum(xm, mp[pl.ds(kk * B, B), :])
        x_mean = pa[:, 288:384] / cnt
        x_att = pa[:, 384:480] / sg
        gr = jnp.concatenate([x_mean, xm, x_att], axis=1)
        gr = jnp.maximum(jnp.dot(gr, wm1[...], preferred_element_type=F32) + cm1[...], 0.0)
        gr = jnp.maximum(jnp.dot(gr, wm2[...], preferred_element_type=F32) + cm2[...], 0.0)
        sh = jnp.maximum(jnp.dot(gr, ws1[...], preferred_element_type=F32) + cs1[...], 0.0)
        sh = jnp.maximum(jnp.dot(sh, ws2[...], preferred_element_type=F32) + cs2[...], 0.0)
        pools = pa[:, 0:384] / cnt
        pj = jnp.maximum(jnp.dot(pools, wpj[...], preferred_element_type=F32) + cpj[...], 0.0)
        f = jnp.dot(sh, wv[...], preferred_element_type=F32) + cv[...]
        f = jnp.dot(f, wo[...], preferred_element_type=F32) + co[...]
        h1 = jnp.maximum(jnp.dot(f, w1[...], preferred_element_type=F32) + c1[...], 0.0)
        h2 = jnp.maximum(jnp.dot(h1, w2[...], preferred_element_type=F32) + c2[...], 0.0)
        zl = jnp.dot(h2, w3[...], preferred_element_type=F32) + c3[...]
        z = 1.0 / (1.0 + jnp.exp(-zl))
        cols = []
        for t in range(3):
            cols.append(z[:, 4 * t:4 * t + 1])
        for t in range(3):
            a = z[:, 4 * t + 1:4 * t + 2]
            bb = z[:, 4 * t + 2:4 * t + 3]
            c = z[:, 4 * t + 3:4 * t + 4]
            m = (a + bb + c) / 3.0
            var = ((a - m) ** 2 + (bb - m) ** 2 + (c - m) ** 2) / 2.0
            unc = jnp.sqrt(var)
            cols.append(m * (1.0 - unc * 0.5))
            cols.append(unc)
        outa[...] = jnp.concatenate(cols + [jnp.zeros((B, 7), F32)], axis=1)
        outs[...] = sh
        outp[...] = pj

    full = lambda shp: pl.BlockSpec(shp, lambda: tuple(0 for _ in shp))
    args = [poolA, poolB, maxparts,
            Wm1, bm1.reshape(1, -1), Wm2, bm2.reshape(1, -1),
            Ws1, bs1.reshape(1, -1), Ws2, bs2.reshape(1, -1),
            Wpj, bpj.reshape(1, -1), Wv, bv.reshape(1, -1),
            Wo, bo.reshape(1, -1), W1, b1.reshape(1, -1),
            W2, b2.reshape(1, -1), W3, b3.reshape(1, -1)]
    return pl.pallas_call(
        body,
        in_specs=[full(a.shape) for a in args],
        out_specs=[full((B, 16)), full((B, H_DIM)), full((B, 512))],
        out_shape=[jax.ShapeDtypeStruct((B, 16), F32),
                   jax.ShapeDtypeStruct((B, H_DIM), F32),
                   jax.ShapeDtypeStruct((B, 512), F32)],
    )(*args)


def _fold_att(lin, att):
    return jnp.einsum("dhc,hc->dh", lin.reshape(H_DIM, HEADS, HC), att)


def kernel(x, edge_index, edge_attr, batch, params, return_hidden):
    src = edge_index[0].astype(I32)
    dst = edge_index[1].astype(I32)

    gats = params["gat"]
    A = [_fold_att(g["lin_edge"], g["att_edge"]) for g in gats]
    S = [_fold_att(g["lin"], g["att_src"]) for g in gats]
    D = [_fold_att(g["lin"], g["att_dst"]) for g in gats]
    We, be = params["edge_emb"]["W"], params["edge_emb"]["b"]
    Wp64 = jnp.zeros((3, 64), F32)
    bp64 = jnp.zeros((64,), F32)
    for l in range(3):
        Wp64 = Wp64.at[:, l * 16:l * 16 + 8].set(We @ A[l])
        bp64 = bp64.at[l * 16:l * 16 + 8].set(be @ A[l])
    bp64 = bp64.at[48].set(1.0)
    Wcat = []
    for l in range(3):
        wc = jnp.zeros((H_DIM, 128), F32)
        wc = wc.at[:, 0:96].set(gats[l]["lin"])
        wc = wc.at[:, 96:104].set(S[l])
        wc = wc.at[:, 112:120].set(D[l])
        Wcat.append(wc)
    zero128 = jnp.zeros((128,), F32)

    xp = jnp.zeros((NP, 9), F32).at[:N].set(x)
    eap = jnp.zeros((EP, 3), F32).at[:E].set(edge_attr)
    padv = jnp.full((EP - E,), N, I32)
    loopi = jnp.arange(NP, dtype=I32)
    tailv = jnp.full((EL - EP - NP,), N, I32)
    srcf = jnp.concatenate([src, padv, loopi, tailv]).reshape(EL // 128, 128)
    dstf = jnp.concatenate([dst, padv, loopi, tailv]).reshape(EL // 128, 128)
    bidx = jnp.concatenate([batch.astype(I32), jnp.full((NP - N,), B, I32)])
    zeros16 = jnp.zeros((NP, 16), F32)
    zeros64 = jnp.zeros((NP, 64), F32)
    zeros96 = jnp.zeros((NP, H_DIM), F32)
    hmap = (jnp.arange(96, dtype=I32) // HC).reshape(6, 16)

    (h0,) = _mm(xp, params["node_emb"]["W"], params["node_emb"]["b"], act=False)
    ae64, aemaxb = _mm(eap, Wp64, bp64, act=False, colmax=True)
    aemax = jnp.max(aemaxb, axis=(0, 1))

    degacc = _sc_scatter_deg(dstf, ae64, zeros64)
    loop64, lmaxb = _tc_degcomb(degacc)
    lmax = jnp.max(lmaxb, axis=(0, 1))

    zpad = jnp.zeros((EL - EP - NP, 16), F32)
    hcur = h0
    residual = h0
    hidden = [h0]
    alphas = []
    for l in range(3):
        zs, zmaxb = _mm(hcur, Wcat[l], zero128, act=False, colmax=True)
        zmax = jnp.max(zmaxb, axis=(0, 1))
        xs = zs[:, 0:96]
        asrc = zs[:, 96:112]
        adst = zs[:, 112:128]
        m_ae = jnp.maximum(aemax[l * 16:l * 16 + 8], lmax[l * 16:l * 16 + 8])
        m8 = zmax[96:104] + zmax[112:120] + m_ae
        m8 = jnp.where(m8 >= 0.0, m8, m8 * 0.2)
        m16 = jnp.concatenate([m8, jnp.zeros((8,), F32)])
        aef = jnp.concatenate(
            [ae64[:, l * 16:(l + 1) * 16], loop64[:, l * 16:(l + 1) * 16], zpad])
        ehbm, sparts = _sc_pass1(srcf, dstf, asrc, adst, aef, m16, zeros16)
        sc = _tc_scomb(sparts)
        ahbm, mparts = _sc_pass2(srcf, dstf, ehbm, sc, xs, hmap, zeros96)
        alphas.append(jnp.concatenate([ahbm[0:E, 0:8], ahbm[EP:EP + N, 0:8]]))
        g = gats[l]
        bn_scale = g["bn_g"] / jnp.sqrt(g["bn_rv"] + 1e-5)
        bn_shift = g["bn_b"] - g["bn_rm"] * bn_scale
        res = residual if l == 2 else None
        hcur = _tc_post(mparts, g["bias"], bn_scale, bn_shift, res)
        hidden.append(hcur)

    (g1,) = _mm(hidden[3], params["gate1"]["W"], params["gate1"]["b"], act=True)
    Wg2 = jnp.zeros((48, 16), F32).at[:, 0].set(params["gate2"]["W"][:, 0])
    bg2 = jnp.zeros((16,), F32).at[0].set(params["gate2"]["b"][0])
    gv, gmaxb = _mm(g1, Wg2, bg2, act=False, colmax=True)
    mg = jnp.max(gmaxb, axis=(0, 1))[0].reshape(1, 1)

    poolA, poolB = _tc_pool(hidden[0], hidden[1], hidden[2], hidden[3],
                            gv, mg, bidx.reshape(NP, 1))
    maxparts = _sc_maxpool(hidden[3], bidx, zeros96).reshape(NW * B, H_DIM)

    Wpj = jnp.zeros((384, 512), F32)
    bpj = jnp.zeros((512,), F32)
    for i in range(4):
        Wpj = Wpj.at[i * 96:(i + 1) * 96, i * 128:(i + 1) * 128].set(
            params["proj"][i]["W"])
        bpj = bpj.at[i * 128:(i + 1) * 128].set(params["proj"][i]["b"])
    mlps = []
    for t in TASKS:
        mlps.append(params["head_" + t])
        mlps.extend(params["conf_" + t])
    W1 = jnp.concatenate([m[0]["W"] for m in mlps], axis=1)
    b1 = jnp.concatenate([m[0]["b"] for m in mlps])
    W2 = jnp.zeros((576, 288), F32)
    b2 = jnp.concatenate([m[1]["b"] for m in mlps])
    W3 = jnp.zeros((288, 16), F32)
    b3 = jnp.zeros((16,), F32)
    for i, m in enumerate(mlps):
        W2 = W2.at[i * 48:(i + 1) * 48, i * 24:(i + 1) * 24].set(m[1]["W"])
        W3 = W3.at[i * 24:(i + 1) * 24, i].set(m[2]["W"][:, 0])
        b3 = b3.at[i].set(m[2]["b"][0])
    mha = params["mha"]

    outa, shared, pj = _tc_readout(
        poolA, poolB, maxparts,
        params["mlp1"]["W"], params["mlp1"]["b"],
        params["mlp2"]["W"], params["mlp2"]["b"],
        params["sf1"]["W"], params["sf1"]["b"],
        params["sf2"]["W"], params["sf2"]["b"],
        Wpj, bpj, mha["Wv"], mha["bv"], mha["Wo"], mha["bo"],
        W1, b1, W2, b2, W3, b3)

    preds = [outa[:, t:t + 1] for t in range(3)]
    confs = [outa[:, 3 + 2 * t:4 + 2 * t] for t in range(3)]
    uncs = [outa[:, 4 + 2 * t:5 + 2 * t] for t in range(3)]
    proj = [pj[:, i * 128:(i + 1) * 128] for i in range(4)]
    return (*preds, *proj, *confs, shared, *alphas, *uncs)
